# Initial kernel scaffold; baseline (speedup 1.0000x reference)
import jax
import jax.numpy as jnp
from jax.experimental import pallas as pl


def kernel(x, edge_index, edge_label_index, edge_attr, W_pre1, b_pre1, W_pre2, b_pre2, W_conv1, b_conv1, W_conv2, b_conv2, wih1, whh1, bih1, bhh1, wih2, whh2, bih2, bhh2, W_post, b_post, prev_emb1, prev_emb2):
    raise NotImplementedError("write your pallas kernel here")


# speedup vs baseline: 17.7993x; 17.7993x over previous
import functools

import jax
import jax.numpy as jnp
from jax import lax
from jax.experimental import pallas as pl
from jax.experimental.pallas import tpu as pltpu
from jax.experimental.pallas import tpu_sc as plsc

N = 10000
E = 320000
NC = 2
NS = 16
NW = NC * NS
EPW = E // NW
C = 80
NCH = EPW // C
NPAD = 10112
RPT = NPAD // NS
DW = 16


@functools.lru_cache(maxsize=None)
def _sc_mesh():
    return plsc.VectorSubcoreMesh(core_axis_name="c", subcore_axis_name="s",
                                  num_cores=NC, num_subcores=NS)


def _leaky(v):
    return jnp.where(v >= 0, v, 0.01 * v)


def _deg_body(dst_hbm, ones_hbm, zeros_hbm, degcnt_hbm, dst_v, ones_v, acc_sh):
    c = lax.axis_index("c")
    s = lax.axis_index("s")
    wid = c * NS + s
    pltpu.sync_copy(dst_hbm.at[wid], dst_v)
    pltpu.sync_copy(ones_hbm, ones_v)
    pltpu.sync_copy(zeros_hbm.at[pl.ds(s * RPT, RPT)],
                    acc_sh.at[pl.ds(s * RPT, RPT)])
    plsc.subcore_barrier()

    def body(j, carry):
        pltpu.sync_copy(ones_v, acc_sh.at[dst_v.at[j]], add=True)
        return carry

    lax.fori_loop(0, NCH, body, 0)
    plsc.subcore_barrier()
    pltpu.sync_copy(acc_sh.at[pl.ds(s * RPT, RPT)],
                    degcnt_hbm.at[c, pl.ds(s * RPT, RPT)])


@functools.lru_cache(maxsize=None)
def _deg_call():
    return pl.kernel(
        _deg_body,
        out_type=jax.ShapeDtypeStruct((NC, NPAD, DW), jnp.float32),
        mesh=_sc_mesh(),
        compiler_params=pltpu.CompilerParams(use_tc_tiling_on_sc=False),
        scratch_types=[
            pltpu.VMEM((NCH, C), jnp.int32),
            pltpu.VMEM((C, DW), jnp.float32),
            pltpu.VMEM_SHARED((NPAD, DW), jnp.float32),
        ],
    )


def _scatter_body(y_hbm, src_hbm, dst_hbm, zeros_hbm, acc_hbm,
                  src_v, dst_v, rows0, rows1, acc_sh, sem0, sem1):
    c = lax.axis_index("c")
    s = lax.axis_index("s")
    wid = c * NS + s
    pltpu.sync_copy(src_hbm.at[wid], src_v)
    pltpu.sync_copy(dst_hbm.at[wid], dst_v)
    pltpu.sync_copy(zeros_hbm.at[pl.ds(s * RPT, RPT)],
                    acc_sh.at[pl.ds(s * RPT, RPT)])
    plsc.subcore_barrier()

    pltpu.async_copy(y_hbm.at[src_v.at[0]], rows0, sem0)

    def body(k, carry):
        j0 = 2 * k
        j1 = j0 + 1
        j2 = j0 + 2
        pltpu.async_copy(y_hbm.at[src_v.at[j1]], rows1, sem1)
        pltpu.make_async_copy(y_hbm.at[src_v.at[j0]], rows0, sem0).wait()
        pltpu.sync_copy(rows0, acc_sh.at[dst_v.at[j0]], add=True)
        pltpu.async_copy(y_hbm.at[src_v.at[j2]], rows0, sem0)
        pltpu.make_async_copy(y_hbm.at[src_v.at[j1]], rows1, sem1).wait()
        pltpu.sync_copy(rows1, acc_sh.at[dst_v.at[j1]], add=True)
        return carry

    lax.fori_loop(0, (NCH - 1) // 2, body, 0)
    pltpu.make_async_copy(y_hbm.at[src_v.at[NCH - 1]], rows0, sem0).wait()
    pltpu.sync_copy(rows0, acc_sh.at[dst_v.at[NCH - 1]], add=True)
    plsc.subcore_barrier()
    pltpu.sync_copy(acc_sh.at[pl.ds(s * RPT, RPT)],
                    acc_hbm.at[c, pl.ds(s * RPT, RPT)])


@functools.lru_cache(maxsize=None)
def _scatter_call():
    return pl.kernel(
        _scatter_body,
        out_type=jax.ShapeDtypeStruct((NC, NPAD, 64), jnp.float32),
        mesh=_sc_mesh(),
        compiler_params=pltpu.CompilerParams(use_tc_tiling_on_sc=False),
        scratch_types=[
            pltpu.VMEM((NCH, C), jnp.int32),
            pltpu.VMEM((NCH, C), jnp.int32),
            pltpu.VMEM((C, 64), jnp.float32),
            pltpu.VMEM((C, 64), jnp.float32),
            pltpu.VMEM_SHARED((NPAD, 64), jnp.float32),
            pltpu.SemaphoreType.DMA,
            pltpu.SemaphoreType.DMA,
        ],
    )


def _edge_body(pq_hbm, e0_hbm, e1_hbm, r_hbm, out_hbm,
               pq_v, e0_v, e1_v, r_v, o_v):
    c = lax.axis_index("c")
    s = lax.axis_index("s")
    wid = c * NS + s
    base = wid * EPW
    pltpu.sync_copy(pq_hbm, pq_v)
    pltpu.sync_copy(e0_hbm.at[pl.ds(base, EPW)], e0_v)
    pltpu.sync_copy(e1_hbm.at[pl.ds(base, EPW)], e1_v)
    pltpu.sync_copy(r_hbm.at[pl.ds(base, EPW)], r_v)

    def body(i, carry):
        sl = pl.ds(i * 16, 16)
        si = e0_v[sl]
        di = e1_v[sl]
        gp = plsc.load_gather(pq_v, [si])
        gq = plsc.load_gather(pq_v, [di + N])
        o_v[sl] = gp + gq + r_v[sl]
        return carry

    lax.fori_loop(0, EPW // 16, body, 0)
    pltpu.sync_copy(o_v, out_hbm.at[pl.ds(base, EPW)])


@functools.lru_cache(maxsize=None)
def _edge_call():
    return pl.kernel(
        _edge_body,
        out_type=jax.ShapeDtypeStruct((E,), jnp.float32),
        mesh=_sc_mesh(),
        compiler_params=pltpu.CompilerParams(needs_layout_passes=False),
        scratch_types=[
            pltpu.VMEM((2 * N,), jnp.float32),
            pltpu.VMEM((EPW,), jnp.int32),
            pltpu.VMEM((EPW,), jnp.int32),
            pltpu.VMEM((EPW,), jnp.float32),
            pltpu.VMEM((EPW,), jnp.float32),
        ],
    )


_RB = 1000
_GRID = N // _RB


def _full(shape):
    nd = len(shape)
    return pl.BlockSpec(shape, lambda i, _n=nd: (0,) * _n)


def _pre_body(x_ref, w1_ref, b1_ref, w2_ref, b2_ref, wclo_ref, wchi_ref,
              olo_ref, ohi_ref):
    h = jnp.dot(x_ref[...], w1_ref[...], preferred_element_type=jnp.float32)
    h = _leaky(h + b1_ref[...])
    h = jnp.dot(h, w2_ref[...], preferred_element_type=jnp.float32)
    h = _leaky(h + b2_ref[...])
    olo_ref[...] = jnp.dot(h, wclo_ref[...], preferred_element_type=jnp.float32)
    ohi_ref[...] = jnp.dot(h, wchi_ref[...], preferred_element_type=jnp.float32)


def _pre_call(x, w1, b1, w2, b2, wclo, wchi):
    return pl.pallas_call(
        _pre_body,
        grid=(_GRID,),
        in_specs=[
            pl.BlockSpec((_RB, 128), lambda i: (i, 0)),
            _full((128, 256)), _full((1, 256)),
            _full((256, 128)), _full((1, 128)),
            _full((128, 64)), _full((128, 64)),
        ],
        out_specs=[
            pl.BlockSpec((_RB, 64), lambda i: (i, 0)),
            pl.BlockSpec((_RB, 64), lambda i: (i, 0)),
        ],
        out_shape=[
            jax.ShapeDtypeStruct((N, 64), jnp.float32),
            jax.ShapeDtypeStruct((N, 64), jnp.float32),
        ],
    )(x, w1, b1, w2, b2, wclo, wchi)


def _dinv_of(deg_blk):
    return lax.rsqrt(1.0 + jnp.sum(deg_blk, axis=(0, 2)) * (1.0 / DW))


def _y1_body(xwlo_ref, xwhi_ref, deg_ref, olo_ref, ohi_ref):
    dinv = _dinv_of(deg_ref[...])
    olo_ref[...] = xwlo_ref[...] * dinv[:, None]
    ohi_ref[...] = xwhi_ref[...] * dinv[:, None]


def _y1_call(xwlo, xwhi, degcnt):
    return pl.pallas_call(
        _y1_body,
        grid=(_GRID,),
        in_specs=[
            pl.BlockSpec((_RB, 64), lambda i: (i, 0)),
            pl.BlockSpec((_RB, 64), lambda i: (i, 0)),
            pl.BlockSpec((NC, _RB, DW), lambda i: (0, i, 0)),
        ],
        out_specs=[
            pl.BlockSpec((_RB, 64), lambda i: (i, 0)),
            pl.BlockSpec((_RB, 64), lambda i: (i, 0)),
        ],
        out_shape=[
            jax.ShapeDtypeStruct((N, 64), jnp.float32),
            jax.ShapeDtypeStruct((N, 64), jnp.float32),
        ],
    )(xwlo, xwhi, degcnt)


def _gru1_body(acclo_ref, acchi_ref, ylo_ref, yhi_ref, deg_ref, prev_ref,
               wihlo_ref, wihhi_ref, whh_ref, bih_ref, bhh_ref,
               bclo_ref, bchi_ref, wc2_ref, emb_ref, y2_ref):
    dinv = _dinv_of(deg_ref[...])
    tlo = _leaky((acclo_ref[0] + acclo_ref[1] + ylo_ref[...])
                 * dinv[:, None] + bclo_ref[...])
    thi = _leaky((acchi_ref[0] + acchi_ref[1] + yhi_ref[...])
                 * dinv[:, None] + bchi_ref[...])
    dn = (((1,), (1,)), ((), ()))
    gi = (lax.dot_general(tlo, wihlo_ref[...], dn,
                          preferred_element_type=jnp.float32)
          + lax.dot_general(thi, wihhi_ref[...], dn,
                            preferred_element_type=jnp.float32)
          + bih_ref[...])
    gh = lax.dot_general(prev_ref[...], whh_ref[...], dn,
                         preferred_element_type=jnp.float32) + bhh_ref[...]
    r = jax.nn.sigmoid(gi[:, 0:128] + gh[:, 0:128])
    z = jax.nn.sigmoid(gi[:, 128:256] + gh[:, 128:256])
    n = jnp.tanh(gi[:, 256:384] + r * gh[:, 256:384])
    e1 = (1.0 - z) * n + z * prev_ref[...]
    emb_ref[...] = e1
    y2_ref[...] = jnp.dot(e1, wc2_ref[...],
                          preferred_element_type=jnp.float32) * dinv[:, None]


def _gru1_call(acclo, acchi, ylo, yhi, degcnt, prev1, wihlo, wihhi, whh1,
               bih1, bhh1, bclo, bchi, wc2):
    return pl.pallas_call(
        _gru1_body,
        grid=(_GRID,),
        in_specs=[
            pl.BlockSpec((NC, _RB, 64), lambda i: (0, i, 0)),
            pl.BlockSpec((NC, _RB, 64), lambda i: (0, i, 0)),
            pl.BlockSpec((_RB, 64), lambda i: (i, 0)),
            pl.BlockSpec((_RB, 64), lambda i: (i, 0)),
            pl.BlockSpec((NC, _RB, DW), lambda i: (0, i, 0)),
            pl.BlockSpec((_RB, 128), lambda i: (i, 0)),
            _full((384, 64)), _full((384, 64)), _full((384, 128)),
            _full((1, 384)), _full((1, 384)),
            _full((1, 64)), _full((1, 64)),
            _full((128, 64)),
        ],
        out_specs=[
            pl.BlockSpec((_RB, 128), lambda i: (i, 0)),
            pl.BlockSpec((_RB, 64), lambda i: (i, 0)),
        ],
        out_shape=[
            jax.ShapeDtypeStruct((N, 128), jnp.float32),
            jax.ShapeDtypeStruct((N, 64), jnp.float32),
        ],
    )(acclo, acchi, ylo, yhi, degcnt, prev1, wihlo, wihhi, whh1,
      bih1, bhh1, bclo, bchi, wc2)


def _gru2_body(acc_ref, y_ref, deg_ref, prev_ref,
               wir_ref, wiz_ref, win_ref, whr_ref, whz_ref, whn_ref,
               bir_ref, biz_ref, bin_ref, bhr_ref, bhz_ref, bhn_ref,
               bc_ref, wpq_ref, emb_ref, pq_ref):
    dinv = _dinv_of(deg_ref[...])
    agg = acc_ref[0] + acc_ref[1] + y_ref[...]
    t = _leaky(agg * dinv[:, None] + bc_ref[...])
    prev = prev_ref[...]
    dn = (((1,), (1,)), ((), ()))
    i_r = lax.dot_general(t, wir_ref[...], dn,
                          preferred_element_type=jnp.float32) + bir_ref[...]
    i_z = lax.dot_general(t, wiz_ref[...], dn,
                          preferred_element_type=jnp.float32) + biz_ref[...]
    i_n = lax.dot_general(t, win_ref[...], dn,
                          preferred_element_type=jnp.float32) + bin_ref[...]
    h_r = lax.dot_general(prev, whr_ref[...], dn,
                          preferred_element_type=jnp.float32) + bhr_ref[...]
    h_z = lax.dot_general(prev, whz_ref[...], dn,
                          preferred_element_type=jnp.float32) + bhz_ref[...]
    h_n = lax.dot_general(prev, whn_ref[...], dn,
                          preferred_element_type=jnp.float32) + bhn_ref[...]
    r = jax.nn.sigmoid(i_r + h_r)
    z = jax.nn.sigmoid(i_z + h_z)
    n = jnp.tanh(i_n + r * h_n)
    e2 = (1.0 - z) * n + z * prev
    emb_ref[...] = e2
    pq_ref[...] = jnp.dot(e2, wpq_ref[...], preferred_element_type=jnp.float32)


def _gru2_call(acc2, y2, degcnt, prev2, wir, wiz, win, whr, whz, whn,
               bir, biz, bin_, bhr, bhz, bhn, bc2, wpq):
    return pl.pallas_call(
        _gru2_body,
        grid=(_GRID,),
        in_specs=[
            pl.BlockSpec((NC, _RB, 64), lambda i: (0, i, 0)),
            pl.BlockSpec((_RB, 64), lambda i: (i, 0)),
            pl.BlockSpec((NC, _RB, DW), lambda i: (0, i, 0)),
            pl.BlockSpec((_RB, 64), lambda i: (i, 0)),
            _full((64, 64)), _full((64, 64)), _full((64, 64)),
            _full((64, 64)), _full((64, 64)), _full((64, 64)),
            _full((1, 64)), _full((1, 64)), _full((1, 64)),
            _full((1, 64)), _full((1, 64)), _full((1, 64)),
            _full((1, 64)), _full((64, 2)),
        ],
        out_specs=[
            pl.BlockSpec((_RB, 64), lambda i: (i, 0)),
            pl.BlockSpec((_RB, 2), lambda i: (i, 0)),
        ],
        out_shape=[
            jax.ShapeDtypeStruct((N, 64), jnp.float32),
            jax.ShapeDtypeStruct((N, 2), jnp.float32),
        ],
    )(acc2, y2, degcnt, prev2, wir, wiz, win, whr, whz, whn,
      bir, biz, bin_, bhr, bhz, bhn, bc2, wpq)


_EB = 4000


def _r_body(attr_ref, wr_ref, bp_ref, o_ref):
    o_ref[...] = jnp.dot(attr_ref[...], wr_ref[...],
                         preferred_element_type=jnp.float32) + bp_ref[...]


def _r_call(edge_attr, wr, bp):
    return pl.pallas_call(
        _r_body,
        grid=(E // _EB,),
        in_specs=[
            pl.BlockSpec((_EB, 16), lambda i: (i, 0)),
            _full((16, 1)), _full((1, 1)),
        ],
        out_specs=pl.BlockSpec((_EB, 1), lambda i: (i, 0)),
        out_shape=jax.ShapeDtypeStruct((E, 1), jnp.float32),
    )(edge_attr, wr, bp)


def kernel(x, edge_index, edge_label_index, edge_attr,
           W_pre1, b_pre1, W_pre2, b_pre2,
           W_conv1, b_conv1, W_conv2, b_conv2,
           wih1, whh1, bih1, bhh1,
           wih2, whh2, bih2, bhh2,
           W_post, b_post, prev_emb1, prev_emb2):
    src3 = edge_index[0].astype(jnp.int32).reshape(NW, NCH, C)
    dst3 = edge_index[1].astype(jnp.int32).reshape(NW, NCH, C)
    e0 = edge_label_index[0].astype(jnp.int32)
    e1 = edge_label_index[1].astype(jnp.int32)

    onesd = jnp.ones((C, DW), jnp.float32)
    zd = jnp.zeros((NPAD, DW), jnp.float32)
    z64 = jnp.zeros((NPAD, 64), jnp.float32)

    xwlo, xwhi = _pre_call(x, W_pre1, b_pre1.reshape(1, -1),
                           W_pre2, b_pre2.reshape(1, -1),
                           W_conv1[:, 0:64], W_conv1[:, 64:128])
    degcnt = _deg_call()(dst3, onesd, zd)
    ylo, yhi = _y1_call(xwlo, xwhi, degcnt)
    scat = _scatter_call()
    acclo = scat(ylo, src3, dst3, z64)
    acchi = scat(yhi, src3, dst3, z64)
    emb1, y2 = _gru1_call(acclo, acchi, ylo, yhi, degcnt, prev_emb1,
                          wih1[:, 0:64], wih1[:, 64:128], whh1,
                          bih1.reshape(1, -1), bhh1.reshape(1, -1),
                          b_conv1[0:64].reshape(1, -1),
                          b_conv1[64:128].reshape(1, -1), W_conv2)
    acc2 = scat(y2, src3, dst3, z64)
    wpq = jnp.concatenate([W_post[0:64], W_post[64:128]], axis=1)
    emb2, pq = _gru2_call(acc2, y2, degcnt, prev_emb2,
                          wih2[0:64], wih2[64:128], wih2[128:192],
                          whh2[0:64], whh2[64:128], whh2[128:192],
                          bih2[0:64].reshape(1, -1),
                          bih2[64:128].reshape(1, -1),
                          bih2[128:192].reshape(1, -1),
                          bhh2[0:64].reshape(1, -1),
                          bhh2[64:128].reshape(1, -1),
                          bhh2[128:192].reshape(1, -1),
                          b_conv2.reshape(1, -1), wpq)
    r = _r_call(edge_attr, W_post[128:144], b_post.reshape(1, 1))
    pqflat = jnp.concatenate([pq[:, 0], pq[:, 1]])
    out = _edge_call()(pqflat, e0, e1, r[:, 0])
    return out, emb1, emb2


# speedup vs baseline: 18.4940x; 1.0390x over previous
import functools

import jax
import jax.numpy as jnp
from jax import lax
from jax.experimental import pallas as pl
from jax.experimental.pallas import tpu as pltpu
from jax.experimental.pallas import tpu_sc as plsc

N = 10000
E = 320000
NC = 2
NS = 16
NW = NC * NS
EPW = E // NW
C = 80
NCH = EPW // C
NPAD = 10112
RPT = NPAD // NS
DW = 16


@functools.lru_cache(maxsize=None)
def _sc_mesh():
    return plsc.VectorSubcoreMesh(core_axis_name="c", subcore_axis_name="s",
                                  num_cores=NC, num_subcores=NS)


def _leaky(v):
    return jnp.where(v >= 0, v, 0.01 * v)


def _deg_body(dst_hbm, ones_hbm, zeros_hbm, degcnt_hbm, dst_v, ones_v, acc_sh):
    c = lax.axis_index("c")
    s = lax.axis_index("s")
    wid = c * NS + s
    pltpu.sync_copy(dst_hbm.at[wid], dst_v)
    pltpu.sync_copy(ones_hbm, ones_v)
    pltpu.sync_copy(zeros_hbm.at[pl.ds(s * RPT, RPT)],
                    acc_sh.at[pl.ds(s * RPT, RPT)])
    plsc.subcore_barrier()

    def body(j, carry):
        pltpu.sync_copy(ones_v, acc_sh.at[dst_v.at[j]], add=True)
        return carry

    lax.fori_loop(0, NCH, body, 0)
    plsc.subcore_barrier()
    pltpu.sync_copy(acc_sh.at[pl.ds(s * RPT, RPT)],
                    degcnt_hbm.at[c, pl.ds(s * RPT, RPT)])


@functools.lru_cache(maxsize=None)
def _deg_call():
    return pl.kernel(
        _deg_body,
        out_type=jax.ShapeDtypeStruct((NC, NPAD, DW), jnp.float32),
        mesh=_sc_mesh(),
        compiler_params=pltpu.CompilerParams(use_tc_tiling_on_sc=False),
        scratch_types=[
            pltpu.VMEM((NCH, C), jnp.int32),
            pltpu.VMEM((C, DW), jnp.float32),
            pltpu.VMEM_SHARED((NPAD, DW), jnp.float32),
        ],
    )


def _scatter_body(y_hbm, src_hbm, dst_hbm, zeros_hbm, acc_hbm,
                  src_v, dst_v, rows0, rows1, acc_sh, sem0, sem1):
    c = lax.axis_index("c")
    s = lax.axis_index("s")
    wid = c * NS + s
    pltpu.sync_copy(src_hbm.at[wid], src_v)
    pltpu.sync_copy(dst_hbm.at[wid], dst_v)
    pltpu.sync_copy(zeros_hbm.at[pl.ds(s * RPT, RPT)],
                    acc_sh.at[pl.ds(s * RPT, RPT)])
    plsc.subcore_barrier()

    pltpu.async_copy(y_hbm.at[src_v.at[0]], rows0, sem0)

    def body(k, carry):
        j0 = 2 * k
        j1 = j0 + 1
        j2 = j0 + 2
        pltpu.async_copy(y_hbm.at[src_v.at[j1]], rows1, sem1)
        pltpu.make_async_copy(y_hbm.at[src_v.at[j0]], rows0, sem0).wait()
        pltpu.sync_copy(rows0, acc_sh.at[dst_v.at[j0]], add=True)
        pltpu.async_copy(y_hbm.at[src_v.at[j2]], rows0, sem0)
        pltpu.make_async_copy(y_hbm.at[src_v.at[j1]], rows1, sem1).wait()
        pltpu.sync_copy(rows1, acc_sh.at[dst_v.at[j1]], add=True)
        return carry

    lax.fori_loop(0, (NCH - 1) // 2, body, 0)
    pltpu.make_async_copy(y_hbm.at[src_v.at[NCH - 1]], rows0, sem0).wait()
    pltpu.sync_copy(rows0, acc_sh.at[dst_v.at[NCH - 1]], add=True)
    plsc.subcore_barrier()
    pltpu.sync_copy(acc_sh.at[pl.ds(s * RPT, RPT)],
                    acc_hbm.at[c, pl.ds(s * RPT, RPT)])


@functools.lru_cache(maxsize=None)
def _scatter_call():
    return pl.kernel(
        _scatter_body,
        out_type=jax.ShapeDtypeStruct((NC, NPAD, 64), jnp.float32),
        mesh=_sc_mesh(),
        compiler_params=pltpu.CompilerParams(use_tc_tiling_on_sc=False),
        scratch_types=[
            pltpu.VMEM((NCH, C), jnp.int32),
            pltpu.VMEM((NCH, C), jnp.int32),
            pltpu.VMEM((C, 64), jnp.float32),
            pltpu.VMEM((C, 64), jnp.float32),
            pltpu.VMEM_SHARED((NPAD, 64), jnp.float32),
            pltpu.SemaphoreType.DMA,
            pltpu.SemaphoreType.DMA,
        ],
    )


def _edge_body(pq_hbm, e0_hbm, e1_hbm, r_hbm, out_hbm,
               pq_v, e0_v, e1_v, r_v, o_v):
    c = lax.axis_index("c")
    s = lax.axis_index("s")
    wid = c * NS + s
    base = wid * EPW
    pltpu.sync_copy(pq_hbm, pq_v)
    pltpu.sync_copy(e0_hbm.at[pl.ds(base, EPW)], e0_v)
    pltpu.sync_copy(e1_hbm.at[pl.ds(base, EPW)], e1_v)
    pltpu.sync_copy(r_hbm.at[pl.ds(base, EPW)], r_v)

    def body(i, carry):
        sl = pl.ds(i * 16, 16)
        si = e0_v[sl]
        di = e1_v[sl]
        gp = plsc.load_gather(pq_v, [si * 2])
        gq = plsc.load_gather(pq_v, [di * 2 + 1])
        o_v[sl] = gp + gq + r_v[sl]
        return carry

    lax.fori_loop(0, EPW // 16, body, 0)
    pltpu.sync_copy(o_v, out_hbm.at[pl.ds(base, EPW)])


@functools.lru_cache(maxsize=None)
def _edge_call():
    return pl.kernel(
        _edge_body,
        out_type=jax.ShapeDtypeStruct((E,), jnp.float32),
        mesh=_sc_mesh(),
        compiler_params=pltpu.CompilerParams(needs_layout_passes=False),
        scratch_types=[
            pltpu.VMEM((2 * N,), jnp.float32),
            pltpu.VMEM((EPW,), jnp.int32),
            pltpu.VMEM((EPW,), jnp.int32),
            pltpu.VMEM((EPW,), jnp.float32),
            pltpu.VMEM((EPW,), jnp.float32),
        ],
    )


_RB = 1000
_GRID = N // _RB


def _full(shape):
    nd = len(shape)
    return pl.BlockSpec(shape, lambda i, _n=nd: (0,) * _n)


def _pre_body(x_ref, w1_ref, b1_ref, w2_ref, b2_ref, wclo_ref, wchi_ref,
              olo_ref, ohi_ref):
    h = jnp.dot(x_ref[...], w1_ref[...], preferred_element_type=jnp.float32)
    h = _leaky(h + b1_ref[...])
    h = jnp.dot(h, w2_ref[...], preferred_element_type=jnp.float32)
    h = _leaky(h + b2_ref[...])
    olo_ref[...] = jnp.dot(h, wclo_ref[...], preferred_element_type=jnp.float32)
    ohi_ref[...] = jnp.dot(h, wchi_ref[...], preferred_element_type=jnp.float32)


def _pre_call(x, w1, b1, w2, b2, wclo, wchi):
    return pl.pallas_call(
        _pre_body,
        grid=(_GRID,),
        in_specs=[
            pl.BlockSpec((_RB, 128), lambda i: (i, 0)),
            _full((128, 256)), _full((1, 256)),
            _full((256, 128)), _full((1, 128)),
            _full((128, 64)), _full((128, 64)),
        ],
        out_specs=[
            pl.BlockSpec((_RB, 64), lambda i: (i, 0)),
            pl.BlockSpec((_RB, 64), lambda i: (i, 0)),
        ],
        out_shape=[
            jax.ShapeDtypeStruct((N, 64), jnp.float32),
            jax.ShapeDtypeStruct((N, 64), jnp.float32),
        ],
    )(x, w1, b1, w2, b2, wclo, wchi)


def _dinv_of(deg_blk):
    return lax.rsqrt(1.0 + jnp.sum(deg_blk, axis=(0, 2)) * (1.0 / DW))


def _y1_body(xwlo_ref, xwhi_ref, deg_ref, olo_ref, ohi_ref):
    dinv = _dinv_of(deg_ref[...])
    olo_ref[...] = xwlo_ref[...] * dinv[:, None]
    ohi_ref[...] = xwhi_ref[...] * dinv[:, None]


def _y1_call(xwlo, xwhi, degcnt):
    return pl.pallas_call(
        _y1_body,
        grid=(_GRID,),
        in_specs=[
            pl.BlockSpec((_RB, 64), lambda i: (i, 0)),
            pl.BlockSpec((_RB, 64), lambda i: (i, 0)),
            pl.BlockSpec((NC, _RB, DW), lambda i: (0, i, 0)),
        ],
        out_specs=[
            pl.BlockSpec((_RB, 64), lambda i: (i, 0)),
            pl.BlockSpec((_RB, 64), lambda i: (i, 0)),
        ],
        out_shape=[
            jax.ShapeDtypeStruct((N, 64), jnp.float32),
            jax.ShapeDtypeStruct((N, 64), jnp.float32),
        ],
    )(xwlo, xwhi, degcnt)


def _gru1_body(acclo_ref, acchi_ref, ylo_ref, yhi_ref, deg_ref, prev_ref,
               wihlo_ref, wihhi_ref, whh_ref, bih_ref, bhh_ref,
               bclo_ref, bchi_ref, wc2_ref, emb_ref, y2_ref):
    dinv = _dinv_of(deg_ref[...])
    tlo = _leaky((acclo_ref[0] + acclo_ref[1] + ylo_ref[...])
                 * dinv[:, None] + bclo_ref[...])
    thi = _leaky((acchi_ref[0] + acchi_ref[1] + yhi_ref[...])
                 * dinv[:, None] + bchi_ref[...])
    dn = (((1,), (1,)), ((), ()))
    gi = (lax.dot_general(tlo, wihlo_ref[...], dn,
                          preferred_element_type=jnp.float32)
          + lax.dot_general(thi, wihhi_ref[...], dn,
                            preferred_element_type=jnp.float32)
          + bih_ref[...])
    gh = lax.dot_general(prev_ref[...], whh_ref[...], dn,
                         preferred_element_type=jnp.float32) + bhh_ref[...]
    r = jax.nn.sigmoid(gi[:, 0:128] + gh[:, 0:128])
    z = jax.nn.sigmoid(gi[:, 128:256] + gh[:, 128:256])
    n = jnp.tanh(gi[:, 256:384] + r * gh[:, 256:384])
    e1 = (1.0 - z) * n + z * prev_ref[...]
    emb_ref[...] = e1
    y2_ref[...] = jnp.dot(e1, wc2_ref[...],
                          preferred_element_type=jnp.float32) * dinv[:, None]


def _gru1_call(acclo, acchi, ylo, yhi, degcnt, prev1, wihlo, wihhi, whh1,
               bih1, bhh1, bclo, bchi, wc2):
    return pl.pallas_call(
        _gru1_body,
        grid=(_GRID,),
        in_specs=[
            pl.BlockSpec((NC, _RB, 64), lambda i: (0, i, 0)),
            pl.BlockSpec((NC, _RB, 64), lambda i: (0, i, 0)),
            pl.BlockSpec((_RB, 64), lambda i: (i, 0)),
            pl.BlockSpec((_RB, 64), lambda i: (i, 0)),
            pl.BlockSpec((NC, _RB, DW), lambda i: (0, i, 0)),
            pl.BlockSpec((_RB, 128), lambda i: (i, 0)),
            _full((384, 64)), _full((384, 64)), _full((384, 128)),
            _full((1, 384)), _full((1, 384)),
            _full((1, 64)), _full((1, 64)),
            _full((128, 64)),
        ],
        out_specs=[
            pl.BlockSpec((_RB, 128), lambda i: (i, 0)),
            pl.BlockSpec((_RB, 64), lambda i: (i, 0)),
        ],
        out_shape=[
            jax.ShapeDtypeStruct((N, 128), jnp.float32),
            jax.ShapeDtypeStruct((N, 64), jnp.float32),
        ],
    )(acclo, acchi, ylo, yhi, degcnt, prev1, wihlo, wihhi, whh1,
      bih1, bhh1, bclo, bchi, wc2)


def _gru2_body(acc_ref, y_ref, deg_ref, prev_ref,
               wir_ref, wiz_ref, win_ref, whr_ref, whz_ref, whn_ref,
               bir_ref, biz_ref, bin_ref, bhr_ref, bhz_ref, bhn_ref,
               bc_ref, wpq_ref, emb_ref, pq_ref):
    dinv = _dinv_of(deg_ref[...])
    agg = acc_ref[0] + acc_ref[1] + y_ref[...]
    t = _leaky(agg * dinv[:, None] + bc_ref[...])
    prev = prev_ref[...]
    dn = (((1,), (1,)), ((), ()))
    i_r = lax.dot_general(t, wir_ref[...], dn,
                          preferred_element_type=jnp.float32) + bir_ref[...]
    i_z = lax.dot_general(t, wiz_ref[...], dn,
                          preferred_element_type=jnp.float32) + biz_ref[...]
    i_n = lax.dot_general(t, win_ref[...], dn,
                          preferred_element_type=jnp.float32) + bin_ref[...]
    h_r = lax.dot_general(prev, whr_ref[...], dn,
                          preferred_element_type=jnp.float32) + bhr_ref[...]
    h_z = lax.dot_general(prev, whz_ref[...], dn,
                          preferred_element_type=jnp.float32) + bhz_ref[...]
    h_n = lax.dot_general(prev, whn_ref[...], dn,
                          preferred_element_type=jnp.float32) + bhn_ref[...]
    r = jax.nn.sigmoid(i_r + h_r)
    z = jax.nn.sigmoid(i_z + h_z)
    n = jnp.tanh(i_n + r * h_n)
    e2 = (1.0 - z) * n + z * prev
    emb_ref[...] = e2
    pq_ref[...] = jnp.dot(e2, wpq_ref[...], preferred_element_type=jnp.float32)


def _gru2_call(acc2, y2, degcnt, prev2, wir, wiz, win, whr, whz, whn,
               bir, biz, bin_, bhr, bhz, bhn, bc2, wpq):
    return pl.pallas_call(
        _gru2_body,
        grid=(_GRID,),
        in_specs=[
            pl.BlockSpec((NC, _RB, 64), lambda i: (0, i, 0)),
            pl.BlockSpec((_RB, 64), lambda i: (i, 0)),
            pl.BlockSpec((NC, _RB, DW), lambda i: (0, i, 0)),
            pl.BlockSpec((_RB, 64), lambda i: (i, 0)),
            _full((64, 64)), _full((64, 64)), _full((64, 64)),
            _full((64, 64)), _full((64, 64)), _full((64, 64)),
            _full((1, 64)), _full((1, 64)), _full((1, 64)),
            _full((1, 64)), _full((1, 64)), _full((1, 64)),
            _full((1, 64)), _full((64, 2)),
        ],
        out_specs=[
            pl.BlockSpec((_RB, 64), lambda i: (i, 0)),
            pl.BlockSpec((_RB, 2), lambda i: (i, 0)),
        ],
        out_shape=[
            jax.ShapeDtypeStruct((N, 64), jnp.float32),
            jax.ShapeDtypeStruct((N, 2), jnp.float32),
        ],
    )(acc2, y2, degcnt, prev2, wir, wiz, win, whr, whz, whn,
      bir, biz, bin_, bhr, bhz, bhn, bc2, wpq)


_EB = 16000


def _r_body(attr_ref, wr_ref, bp_ref, o_ref):
    o_ref[...] = jnp.dot(attr_ref[...], wr_ref[...],
                         preferred_element_type=jnp.float32) + bp_ref[...]


def _r_call(edge_attr, wr, bp):
    return pl.pallas_call(
        _r_body,
        grid=(E // _EB,),
        in_specs=[
            pl.BlockSpec((_EB, 16), lambda i: (i, 0)),
            _full((16, 1)), _full((1, 1)),
        ],
        out_specs=pl.BlockSpec((_EB, 1), lambda i: (i, 0)),
        out_shape=jax.ShapeDtypeStruct((E, 1), jnp.float32),
    )(edge_attr, wr, bp)


def kernel(x, edge_index, edge_label_index, edge_attr,
           W_pre1, b_pre1, W_pre2, b_pre2,
           W_conv1, b_conv1, W_conv2, b_conv2,
           wih1, whh1, bih1, bhh1,
           wih2, whh2, bih2, bhh2,
           W_post, b_post, prev_emb1, prev_emb2):
    src3 = edge_index[0].astype(jnp.int32).reshape(NW, NCH, C)
    dst3 = edge_index[1].astype(jnp.int32).reshape(NW, NCH, C)
    e0 = edge_label_index[0].astype(jnp.int32)
    e1 = edge_label_index[1].astype(jnp.int32)

    onesd = jnp.ones((C, DW), jnp.float32)
    zd = jnp.zeros((NPAD, DW), jnp.float32)
    z64 = jnp.zeros((NPAD, 64), jnp.float32)

    xwlo, xwhi = _pre_call(x, W_pre1, b_pre1.reshape(1, -1),
                           W_pre2, b_pre2.reshape(1, -1),
                           W_conv1[:, 0:64], W_conv1[:, 64:128])
    degcnt = _deg_call()(dst3, onesd, zd)
    ylo, yhi = _y1_call(xwlo, xwhi, degcnt)
    scat = _scatter_call()
    acclo = scat(ylo, src3, dst3, z64)
    acchi = scat(yhi, src3, dst3, z64)
    emb1, y2 = _gru1_call(acclo, acchi, ylo, yhi, degcnt, prev_emb1,
                          wih1[:, 0:64], wih1[:, 64:128], whh1,
                          bih1.reshape(1, -1), bhh1.reshape(1, -1),
                          b_conv1[0:64].reshape(1, -1),
                          b_conv1[64:128].reshape(1, -1), W_conv2)
    acc2 = scat(y2, src3, dst3, z64)
    wpq = jnp.concatenate([W_post[0:64], W_post[64:128]], axis=1)
    emb2, pq = _gru2_call(acc2, y2, degcnt, prev_emb2,
                          wih2[0:64], wih2[64:128], wih2[128:192],
                          whh2[0:64], whh2[64:128], whh2[128:192],
                          bih2[0:64].reshape(1, -1),
                          bih2[64:128].reshape(1, -1),
                          bih2[128:192].reshape(1, -1),
                          bhh2[0:64].reshape(1, -1),
                          bhh2[64:128].reshape(1, -1),
                          bhh2[128:192].reshape(1, -1),
                          b_conv2.reshape(1, -1), wpq)
    r = _r_call(edge_attr, W_post[128:144], b_post.reshape(1, 1))
    out = _edge_call()(pq.reshape(2 * N), e0, e1, r.reshape(E))
    return out, emb1, emb2


# speedup vs baseline: 21.5795x; 1.1668x over previous
import functools

import jax
import jax.numpy as jnp
from jax import lax
from jax.experimental import pallas as pl
from jax.experimental.pallas import tpu as pltpu
from jax.experimental.pallas import tpu_sc as plsc

N = 10000
E = 320000
NC = 2
NS = 16
NW = NC * NS
EPW = E // NW
C = 80
NCH = EPW // C
NPAD = 10112
RPT = NPAD // NS
DW = 16


@functools.lru_cache(maxsize=None)
def _sc_mesh():
    return plsc.VectorSubcoreMesh(core_axis_name="c", subcore_axis_name="s",
                                  num_cores=NC, num_subcores=NS)


def _leaky(v):
    return jnp.where(v >= 0, v, 0.01 * v)


def _deg_body(dst_hbm, ones_hbm, zeros_hbm, degcnt_hbm, dst_v, ones_v, acc_sh):
    c = lax.axis_index("c")
    s = lax.axis_index("s")
    wid = c * NS + s
    pltpu.sync_copy(dst_hbm.at[wid], dst_v)
    pltpu.sync_copy(ones_hbm, ones_v)
    pltpu.sync_copy(zeros_hbm.at[pl.ds(s * RPT, RPT)],
                    acc_sh.at[pl.ds(s * RPT, RPT)])
    plsc.subcore_barrier()

    def body(j, carry):
        pltpu.sync_copy(ones_v, acc_sh.at[dst_v.at[j]], add=True)
        return carry

    lax.fori_loop(0, NCH, body, 0)
    plsc.subcore_barrier()
    pltpu.sync_copy(acc_sh.at[pl.ds(s * RPT, RPT)],
                    degcnt_hbm.at[c, pl.ds(s * RPT, RPT)])


@functools.lru_cache(maxsize=None)
def _deg_call():
    return pl.kernel(
        _deg_body,
        out_type=jax.ShapeDtypeStruct((NC, NPAD, DW), jnp.float32),
        mesh=_sc_mesh(),
        compiler_params=pltpu.CompilerParams(use_tc_tiling_on_sc=False),
        scratch_types=[
            pltpu.VMEM((NCH, C), jnp.int32),
            pltpu.VMEM((C, DW), jnp.float32),
            pltpu.VMEM_SHARED((NPAD, DW), jnp.float32),
        ],
    )


def _scatter_body(y_hbm, src_hbm, dst_hbm, zeros_hbm, acc_hbm,
                  src_v, dst_v, rows0, rows1, acc_sh, sem0, sem1):
    c = lax.axis_index("c")
    s = lax.axis_index("s")
    wid = c * NS + s
    pltpu.sync_copy(src_hbm.at[wid], src_v)
    pltpu.sync_copy(dst_hbm.at[wid], dst_v)
    pltpu.sync_copy(zeros_hbm.at[pl.ds(s * RPT, RPT)],
                    acc_sh.at[pl.ds(s * RPT, RPT)])
    plsc.subcore_barrier()

    pltpu.async_copy(y_hbm.at[src_v.at[0]], rows0, sem0)

    def body(k, carry):
        j0 = 2 * k
        j1 = j0 + 1
        j2 = j0 + 2
        pltpu.async_copy(y_hbm.at[src_v.at[j1]], rows1, sem1)
        pltpu.make_async_copy(y_hbm.at[src_v.at[j0]], rows0, sem0).wait()
        pltpu.sync_copy(rows0, acc_sh.at[dst_v.at[j0]], add=True)
        pltpu.async_copy(y_hbm.at[src_v.at[j2]], rows0, sem0)
        pltpu.make_async_copy(y_hbm.at[src_v.at[j1]], rows1, sem1).wait()
        pltpu.sync_copy(rows1, acc_sh.at[dst_v.at[j1]], add=True)
        return carry

    lax.fori_loop(0, (NCH - 1) // 2, body, 0)
    pltpu.make_async_copy(y_hbm.at[src_v.at[NCH - 1]], rows0, sem0).wait()
    pltpu.sync_copy(rows0, acc_sh.at[dst_v.at[NCH - 1]], add=True)
    plsc.subcore_barrier()
    pltpu.sync_copy(acc_sh.at[pl.ds(s * RPT, RPT)],
                    acc_hbm.at[c, pl.ds(s * RPT, RPT)])


@functools.lru_cache(maxsize=None)
def _scatter_call():
    return pl.kernel(
        _scatter_body,
        out_type=jax.ShapeDtypeStruct((NC, NPAD, 64), jnp.float32),
        mesh=_sc_mesh(),
        compiler_params=pltpu.CompilerParams(use_tc_tiling_on_sc=False),
        scratch_types=[
            pltpu.VMEM((NCH, C), jnp.int32),
            pltpu.VMEM((NCH, C), jnp.int32),
            pltpu.VMEM((C, 64), jnp.float32),
            pltpu.VMEM((C, 64), jnp.float32),
            pltpu.VMEM_SHARED((NPAD, 64), jnp.float32),
            pltpu.SemaphoreType.DMA,
            pltpu.SemaphoreType.DMA,
        ],
    )


def _edge_body(pq_hbm, e0_hbm, e1_hbm, r_hbm, out_hbm,
               pq_v, e0_v, e1_v, r_v, o_v):
    c = lax.axis_index("c")
    s = lax.axis_index("s")
    wid = c * NS + s
    base = wid * EPW
    pltpu.sync_copy(pq_hbm, pq_v)
    pltpu.sync_copy(e0_hbm.at[pl.ds(base, EPW)], e0_v)
    pltpu.sync_copy(e1_hbm.at[pl.ds(base, EPW)], e1_v)
    pltpu.sync_copy(r_hbm.at[pl.ds(base, EPW)], r_v)

    def body(i, carry):
        sl = pl.ds(i * 16, 16)
        si = e0_v[sl]
        di = e1_v[sl]
        gp = plsc.load_gather(pq_v, [si * 2])
        gq = plsc.load_gather(pq_v, [di * 2 + 1])
        o_v[sl] = gp + gq + r_v[sl]
        return carry

    lax.fori_loop(0, EPW // 16, body, 0)
    pltpu.sync_copy(o_v, out_hbm.at[pl.ds(base, EPW)])


@functools.lru_cache(maxsize=None)
def _edge_call():
    return pl.kernel(
        _edge_body,
        out_type=jax.ShapeDtypeStruct((E,), jnp.float32),
        mesh=_sc_mesh(),
        compiler_params=pltpu.CompilerParams(needs_layout_passes=False),
        scratch_types=[
            pltpu.VMEM((2 * N,), jnp.float32),
            pltpu.VMEM((EPW,), jnp.int32),
            pltpu.VMEM((EPW,), jnp.int32),
            pltpu.VMEM((EPW,), jnp.float32),
            pltpu.VMEM((EPW,), jnp.float32),
        ],
    )


_RB = 1000
_GRID = N // _RB


def _full(shape):
    nd = len(shape)
    return pl.BlockSpec(shape, lambda i, _n=nd: (0,) * _n)


def _pre_body(x_ref, w1_ref, b1_ref, w2_ref, b2_ref, wclo_ref, wchi_ref,
              olo_ref, ohi_ref):
    h = jnp.dot(x_ref[...], w1_ref[...], preferred_element_type=jnp.float32)
    h = _leaky(h + b1_ref[...])
    h = jnp.dot(h, w2_ref[...], preferred_element_type=jnp.float32)
    h = _leaky(h + b2_ref[...])
    olo_ref[...] = jnp.dot(h, wclo_ref[...], preferred_element_type=jnp.float32)
    ohi_ref[...] = jnp.dot(h, wchi_ref[...], preferred_element_type=jnp.float32)


def _pre_call(x, w1, b1, w2, b2, wclo, wchi):
    return pl.pallas_call(
        _pre_body,
        grid=(_GRID,),
        in_specs=[
            pl.BlockSpec((_RB, 128), lambda i: (i, 0)),
            _full((128, 256)), _full((1, 256)),
            _full((256, 128)), _full((1, 128)),
            _full((128, 64)), _full((128, 64)),
        ],
        out_specs=[
            pl.BlockSpec((_RB, 64), lambda i: (i, 0)),
            pl.BlockSpec((_RB, 64), lambda i: (i, 0)),
        ],
        out_shape=[
            jax.ShapeDtypeStruct((N, 64), jnp.float32),
            jax.ShapeDtypeStruct((N, 64), jnp.float32),
        ],
    )(x, w1, b1, w2, b2, wclo, wchi)


def _dinv_of(deg_blk):
    return lax.rsqrt(1.0 + jnp.sum(deg_blk, axis=(0, 2)) * (1.0 / DW))


def _y1_body(xwlo_ref, xwhi_ref, deg_ref, olo_ref, ohi_ref):
    dinv = _dinv_of(deg_ref[...])
    olo_ref[...] = xwlo_ref[...] * dinv[:, None]
    ohi_ref[...] = xwhi_ref[...] * dinv[:, None]


def _y1_call(xwlo, xwhi, degcnt):
    return pl.pallas_call(
        _y1_body,
        grid=(_GRID,),
        in_specs=[
            pl.BlockSpec((_RB, 64), lambda i: (i, 0)),
            pl.BlockSpec((_RB, 64), lambda i: (i, 0)),
            pl.BlockSpec((NC, _RB, DW), lambda i: (0, i, 0)),
        ],
        out_specs=[
            pl.BlockSpec((_RB, 64), lambda i: (i, 0)),
            pl.BlockSpec((_RB, 64), lambda i: (i, 0)),
        ],
        out_shape=[
            jax.ShapeDtypeStruct((N, 64), jnp.float32),
            jax.ShapeDtypeStruct((N, 64), jnp.float32),
        ],
    )(xwlo, xwhi, degcnt)


def _gru1_body(acclo_ref, acchi_ref, ylo_ref, yhi_ref, deg_ref, prev_ref,
               wihlo_ref, wihhi_ref, whh_ref, bih_ref, bhh_ref,
               bclo_ref, bchi_ref, wc2_ref, emb_ref, y2_ref):
    dinv = _dinv_of(deg_ref[...])
    tlo = _leaky((acclo_ref[0] + acclo_ref[1] + ylo_ref[...])
                 * dinv[:, None] + bclo_ref[...])
    thi = _leaky((acchi_ref[0] + acchi_ref[1] + yhi_ref[...])
                 * dinv[:, None] + bchi_ref[...])
    dn = (((1,), (1,)), ((), ()))
    gi = (lax.dot_general(tlo, wihlo_ref[...], dn,
                          preferred_element_type=jnp.float32)
          + lax.dot_general(thi, wihhi_ref[...], dn,
                            preferred_element_type=jnp.float32)
          + bih_ref[...])
    gh = lax.dot_general(prev_ref[...], whh_ref[...], dn,
                         preferred_element_type=jnp.float32) + bhh_ref[...]
    r = jax.nn.sigmoid(gi[:, 0:128] + gh[:, 0:128])
    z = jax.nn.sigmoid(gi[:, 128:256] + gh[:, 128:256])
    n = jnp.tanh(gi[:, 256:384] + r * gh[:, 256:384])
    e1 = (1.0 - z) * n + z * prev_ref[...]
    emb_ref[...] = e1
    y2_ref[...] = jnp.dot(e1, wc2_ref[...],
                          preferred_element_type=jnp.float32) * dinv[:, None]


def _gru1_call(acclo, acchi, ylo, yhi, degcnt, prev1, wihlo, wihhi, whh1,
               bih1, bhh1, bclo, bchi, wc2):
    return pl.pallas_call(
        _gru1_body,
        grid=(_GRID,),
        in_specs=[
            pl.BlockSpec((NC, _RB, 64), lambda i: (0, i, 0)),
            pl.BlockSpec((NC, _RB, 64), lambda i: (0, i, 0)),
            pl.BlockSpec((_RB, 64), lambda i: (i, 0)),
            pl.BlockSpec((_RB, 64), lambda i: (i, 0)),
            pl.BlockSpec((NC, _RB, DW), lambda i: (0, i, 0)),
            pl.BlockSpec((_RB, 128), lambda i: (i, 0)),
            _full((384, 64)), _full((384, 64)), _full((384, 128)),
            _full((1, 384)), _full((1, 384)),
            _full((1, 64)), _full((1, 64)),
            _full((128, 64)),
        ],
        out_specs=[
            pl.BlockSpec((_RB, 128), lambda i: (i, 0)),
            pl.BlockSpec((_RB, 64), lambda i: (i, 0)),
        ],
        out_shape=[
            jax.ShapeDtypeStruct((N, 128), jnp.float32),
            jax.ShapeDtypeStruct((N, 64), jnp.float32),
        ],
    )(acclo, acchi, ylo, yhi, degcnt, prev1, wihlo, wihhi, whh1,
      bih1, bhh1, bclo, bchi, wc2)


def _gru2_body(acc_ref, y_ref, deg_ref, prev_ref,
               wir_ref, wiz_ref, win_ref, whr_ref, whz_ref, whn_ref,
               bir_ref, biz_ref, bin_ref, bhr_ref, bhz_ref, bhn_ref,
               bc_ref, wpq_ref, emb_ref, pq_ref):
    dinv = _dinv_of(deg_ref[...])
    agg = acc_ref[0] + acc_ref[1] + y_ref[...]
    t = _leaky(agg * dinv[:, None] + bc_ref[...])
    prev = prev_ref[...]
    dn = (((1,), (1,)), ((), ()))
    i_r = lax.dot_general(t, wir_ref[...], dn,
                          preferred_element_type=jnp.float32) + bir_ref[...]
    i_z = lax.dot_general(t, wiz_ref[...], dn,
                          preferred_element_type=jnp.float32) + biz_ref[...]
    i_n = lax.dot_general(t, win_ref[...], dn,
                          preferred_element_type=jnp.float32) + bin_ref[...]
    h_r = lax.dot_general(prev, whr_ref[...], dn,
                          preferred_element_type=jnp.float32) + bhr_ref[...]
    h_z = lax.dot_general(prev, whz_ref[...], dn,
                          preferred_element_type=jnp.float32) + bhz_ref[...]
    h_n = lax.dot_general(prev, whn_ref[...], dn,
                          preferred_element_type=jnp.float32) + bhn_ref[...]
    r = jax.nn.sigmoid(i_r + h_r)
    z = jax.nn.sigmoid(i_z + h_z)
    n = jnp.tanh(i_n + r * h_n)
    e2 = (1.0 - z) * n + z * prev
    emb_ref[...] = e2
    pq_ref[...] = jnp.dot(e2, wpq_ref[...], preferred_element_type=jnp.float32)


def _gru2_call(acc2, y2, degcnt, prev2, wir, wiz, win, whr, whz, whn,
               bir, biz, bin_, bhr, bhz, bhn, bc2, wpq):
    return pl.pallas_call(
        _gru2_body,
        grid=(_GRID,),
        in_specs=[
            pl.BlockSpec((NC, _RB, 64), lambda i: (0, i, 0)),
            pl.BlockSpec((_RB, 64), lambda i: (i, 0)),
            pl.BlockSpec((NC, _RB, DW), lambda i: (0, i, 0)),
            pl.BlockSpec((_RB, 64), lambda i: (i, 0)),
            _full((64, 64)), _full((64, 64)), _full((64, 64)),
            _full((64, 64)), _full((64, 64)), _full((64, 64)),
            _full((1, 64)), _full((1, 64)), _full((1, 64)),
            _full((1, 64)), _full((1, 64)), _full((1, 64)),
            _full((1, 64)), _full((64, 2)),
        ],
        out_specs=[
            pl.BlockSpec((_RB, 64), lambda i: (i, 0)),
            pl.BlockSpec((_RB, 2), lambda i: (i, 0)),
        ],
        out_shape=[
            jax.ShapeDtypeStruct((N, 64), jnp.float32),
            jax.ShapeDtypeStruct((N, 2), jnp.float32),
        ],
    )(acc2, y2, degcnt, prev2, wir, wiz, win, whr, whz, whn,
      bir, biz, bin_, bhr, bhz, bhn, bc2, wpq)


_ERows = E // 128
_ERB = _ERows // 10


def _r_body(attr_ref, wr_ref, bp_ref, o_ref):
    o_ref[...] = jnp.dot(attr_ref[...], wr_ref[...],
                         preferred_element_type=jnp.float32) + bp_ref[...]


def _r_call(attr128, wr_kron, bp):
    return pl.pallas_call(
        _r_body,
        grid=(1,),
        in_specs=[
            pl.BlockSpec((_ERows, 2048), lambda i: (0, 0)),
            _full((2048, 128)), _full((1, 1)),
        ],
        out_specs=pl.BlockSpec((_ERows, 128), lambda i: (0, 0)),
        out_shape=jax.ShapeDtypeStruct((_ERows, 128), jnp.float32),
    )(attr128, wr_kron, bp)


def kernel(x, edge_index, edge_label_index, edge_attr,
           W_pre1, b_pre1, W_pre2, b_pre2,
           W_conv1, b_conv1, W_conv2, b_conv2,
           wih1, whh1, bih1, bhh1,
           wih2, whh2, bih2, bhh2,
           W_post, b_post, prev_emb1, prev_emb2):
    src3 = edge_index[0].astype(jnp.int32).reshape(NW, NCH, C)
    dst3 = edge_index[1].astype(jnp.int32).reshape(NW, NCH, C)
    e0 = edge_label_index[0].astype(jnp.int32)
    e1 = edge_label_index[1].astype(jnp.int32)

    onesd = jnp.ones((C, DW), jnp.float32)
    zd = jnp.zeros((NPAD, DW), jnp.float32)
    z64 = jnp.zeros((NPAD, 64), jnp.float32)

    xwlo, xwhi = _pre_call(x, W_pre1, b_pre1.reshape(1, -1),
                           W_pre2, b_pre2.reshape(1, -1),
                           W_conv1[:, 0:64], W_conv1[:, 64:128])
    degcnt = _deg_call()(dst3, onesd, zd)
    ylo, yhi = _y1_call(xwlo, xwhi, degcnt)
    scat = _scatter_call()
    acclo = scat(ylo, src3, dst3, z64)
    acchi = scat(yhi, src3, dst3, z64)
    emb1, y2 = _gru1_call(acclo, acchi, ylo, yhi, degcnt, prev_emb1,
                          wih1[:, 0:64], wih1[:, 64:128], whh1,
                          bih1.reshape(1, -1), bhh1.reshape(1, -1),
                          b_conv1[0:64].reshape(1, -1),
                          b_conv1[64:128].reshape(1, -1), W_conv2)
    acc2 = scat(y2, src3, dst3, z64)
    wpq = jnp.concatenate([W_post[0:64], W_post[64:128]], axis=1)
    emb2, pq = _gru2_call(acc2, y2, degcnt, prev_emb2,
                          wih2[0:64], wih2[64:128], wih2[128:192],
                          whh2[0:64], whh2[64:128], whh2[128:192],
                          bih2[0:64].reshape(1, -1),
                          bih2[64:128].reshape(1, -1),
                          bih2[128:192].reshape(1, -1),
                          bhh2[0:64].reshape(1, -1),
                          bhh2[64:128].reshape(1, -1),
                          bhh2[128:192].reshape(1, -1),
                          b_conv2.reshape(1, -1), wpq)
    wr_kron = jnp.kron(jnp.eye(128, dtype=jnp.float32), W_post[128:144])
    r = _r_call(edge_attr.reshape(_ERows, 2048), wr_kron,
                b_post.reshape(1, 1))
    out = _edge_call()(pq.reshape(2 * N), e0, e1, r.reshape(E))
    return out, emb1, emb2


# speedup vs baseline: 23.4592x; 1.0871x over previous
import functools

import jax
import jax.numpy as jnp
from jax import lax
from jax.experimental import pallas as pl
from jax.experimental.pallas import tpu as pltpu
from jax.experimental.pallas import tpu_sc as plsc

N = 10000
E = 320000
NC = 2
NS = 16
NW = NC * NS
EPW = E // NW
C = 80
NCH = EPW // C
NPAD = 10112
RPT = NPAD // NS
DW = 16


@functools.lru_cache(maxsize=None)
def _sc_mesh():
    return plsc.VectorSubcoreMesh(core_axis_name="c", subcore_axis_name="s",
                                  num_cores=NC, num_subcores=NS)


def _leaky(v):
    return jnp.where(v >= 0, v, 0.01 * v)


def _deg_body(dst_hbm, ones_hbm, zeros_hbm, degcnt_hbm, dst_v, ones_v, acc_sh):
    c = lax.axis_index("c")
    s = lax.axis_index("s")
    wid = c * NS + s
    pltpu.sync_copy(dst_hbm.at[wid], dst_v)
    pltpu.sync_copy(ones_hbm, ones_v)
    pltpu.sync_copy(zeros_hbm.at[pl.ds(s * RPT, RPT)],
                    acc_sh.at[pl.ds(s * RPT, RPT)])
    plsc.subcore_barrier()

    def body(j, carry):
        pltpu.sync_copy(ones_v, acc_sh.at[dst_v.at[j]], add=True)
        return carry

    lax.fori_loop(0, NCH, body, 0)
    plsc.subcore_barrier()
    pltpu.sync_copy(acc_sh.at[pl.ds(s * RPT, RPT)],
                    degcnt_hbm.at[c, pl.ds(s * RPT, RPT)])


@functools.lru_cache(maxsize=None)
def _deg_call():
    return pl.kernel(
        _deg_body,
        out_type=jax.ShapeDtypeStruct((NC, NPAD, DW), jnp.float32),
        mesh=_sc_mesh(),
        compiler_params=pltpu.CompilerParams(use_tc_tiling_on_sc=False),
        scratch_types=[
            pltpu.VMEM((NCH, C), jnp.int32),
            pltpu.VMEM((C, DW), jnp.float32),
            pltpu.VMEM_SHARED((NPAD, DW), jnp.float32),
        ],
    )


def _scatter_body(y_hbm, src_hbm, dst_hbm, zeros_hbm, acc_hbm,
                  src_v, dst_v, rows0, rows1, acc_sh, sem0, sem1):
    c = lax.axis_index("c")
    s = lax.axis_index("s")
    wid = c * NS + s
    pltpu.sync_copy(src_hbm.at[wid], src_v)
    pltpu.sync_copy(dst_hbm.at[wid], dst_v)
    pltpu.sync_copy(zeros_hbm.at[pl.ds(s * RPT, RPT)],
                    acc_sh.at[pl.ds(s * RPT, RPT)])
    plsc.subcore_barrier()

    pltpu.async_copy(y_hbm.at[src_v.at[0]], rows0, sem0)

    def body(k, carry):
        j0 = 2 * k
        j1 = j0 + 1
        j2 = j0 + 2
        pltpu.async_copy(y_hbm.at[src_v.at[j1]], rows1, sem1)
        pltpu.make_async_copy(y_hbm.at[src_v.at[j0]], rows0, sem0).wait()
        pltpu.sync_copy(rows0, acc_sh.at[dst_v.at[j0]], add=True)
        pltpu.async_copy(y_hbm.at[src_v.at[j2]], rows0, sem0)
        pltpu.make_async_copy(y_hbm.at[src_v.at[j1]], rows1, sem1).wait()
        pltpu.sync_copy(rows1, acc_sh.at[dst_v.at[j1]], add=True)
        return carry

    lax.fori_loop(0, (NCH - 1) // 2, body, 0)
    pltpu.make_async_copy(y_hbm.at[src_v.at[NCH - 1]], rows0, sem0).wait()
    pltpu.sync_copy(rows0, acc_sh.at[dst_v.at[NCH - 1]], add=True)
    plsc.subcore_barrier()
    pltpu.sync_copy(acc_sh.at[pl.ds(s * RPT, RPT)],
                    acc_hbm.at[c, pl.ds(s * RPT, RPT)])


@functools.lru_cache(maxsize=None)
def _scatter_call():
    return pl.kernel(
        _scatter_body,
        out_type=jax.ShapeDtypeStruct((NC, NPAD, 64), jnp.float32),
        mesh=_sc_mesh(),
        compiler_params=pltpu.CompilerParams(use_tc_tiling_on_sc=False),
        scratch_types=[
            pltpu.VMEM((NCH, C), jnp.int32),
            pltpu.VMEM((NCH, C), jnp.int32),
            pltpu.VMEM((C, 64), jnp.float32),
            pltpu.VMEM((C, 64), jnp.float32),
            pltpu.VMEM_SHARED((NPAD, 64), jnp.float32),
            pltpu.SemaphoreType.DMA,
            pltpu.SemaphoreType.DMA,
        ],
    )


def _edge_body(pq_hbm, e0_hbm, e1_hbm, r_hbm, out_hbm,
               pq_v, e0_v, e1_v, r_v, o_v):
    c = lax.axis_index("c")
    s = lax.axis_index("s")
    wid = c * NS + s
    base = wid * EPW
    pltpu.sync_copy(pq_hbm, pq_v)
    pltpu.sync_copy(e0_hbm.at[pl.ds(base, EPW)], e0_v)
    pltpu.sync_copy(e1_hbm.at[pl.ds(base, EPW)], e1_v)
    pltpu.sync_copy(r_hbm.at[pl.ds(base, EPW)], r_v)

    def body(i, carry):
        sl = pl.ds(i * 16, 16)
        si = e0_v[sl]
        di = e1_v[sl]
        gp = plsc.load_gather(pq_v, [si * 2])
        gq = plsc.load_gather(pq_v, [di * 2 + 1])
        o_v[sl] = gp + gq + r_v[sl]
        return carry

    lax.fori_loop(0, EPW // 16, body, 0)
    pltpu.sync_copy(o_v, out_hbm.at[pl.ds(base, EPW)])


@functools.lru_cache(maxsize=None)
def _edge_call():
    return pl.kernel(
        _edge_body,
        out_type=jax.ShapeDtypeStruct((E,), jnp.float32),
        mesh=_sc_mesh(),
        compiler_params=pltpu.CompilerParams(needs_layout_passes=False),
        scratch_types=[
            pltpu.VMEM((2 * N,), jnp.float32),
            pltpu.VMEM((EPW,), jnp.int32),
            pltpu.VMEM((EPW,), jnp.int32),
            pltpu.VMEM((EPW,), jnp.float32),
            pltpu.VMEM((EPW,), jnp.float32),
        ],
    )


_RB = 1000
_GRID = N // _RB


def _full(shape):
    nd = len(shape)
    return pl.BlockSpec(shape, lambda i, _n=nd: (0,) * _n)


def _pre_body(x_ref, w1_ref, b1_ref, w2_ref, b2_ref, wclo_ref, wchi_ref,
              olo_ref, ohi_ref):
    h = jnp.dot(x_ref[...], w1_ref[...], preferred_element_type=jnp.float32)
    h = _leaky(h + b1_ref[...])
    h = jnp.dot(h, w2_ref[...], preferred_element_type=jnp.float32)
    h = _leaky(h + b2_ref[...])
    olo_ref[...] = jnp.dot(h, wclo_ref[...], preferred_element_type=jnp.float32)
    ohi_ref[...] = jnp.dot(h, wchi_ref[...], preferred_element_type=jnp.float32)


def _pre_call(x, w1, b1, w2, b2, wclo, wchi):
    return pl.pallas_call(
        _pre_body,
        grid=(_GRID,),
        in_specs=[
            pl.BlockSpec((_RB, 128), lambda i: (i, 0)),
            _full((128, 256)), _full((1, 256)),
            _full((256, 128)), _full((1, 128)),
            _full((128, 64)), _full((128, 64)),
        ],
        out_specs=[
            pl.BlockSpec((_RB, 64), lambda i: (i, 0)),
            pl.BlockSpec((_RB, 64), lambda i: (i, 0)),
        ],
        out_shape=[
            jax.ShapeDtypeStruct((N, 64), jnp.float32),
            jax.ShapeDtypeStruct((N, 64), jnp.float32),
        ],
    )(x, w1, b1, w2, b2, wclo, wchi)


def _dinv_of(deg_blk):
    return lax.rsqrt(1.0 + jnp.sum(deg_blk, axis=(0, 2)) * (1.0 / DW))


def _y1_body(xwlo_ref, xwhi_ref, deg_ref, olo_ref, ohi_ref):
    dinv = _dinv_of(deg_ref[...])
    olo_ref[...] = xwlo_ref[...] * dinv[:, None]
    ohi_ref[...] = xwhi_ref[...] * dinv[:, None]


def _y1_call(xwlo, xwhi, degcnt):
    return pl.pallas_call(
        _y1_body,
        grid=(_GRID,),
        in_specs=[
            pl.BlockSpec((_RB, 64), lambda i: (i, 0)),
            pl.BlockSpec((_RB, 64), lambda i: (i, 0)),
            pl.BlockSpec((NC, _RB, DW), lambda i: (0, i, 0)),
        ],
        out_specs=[
            pl.BlockSpec((_RB, 64), lambda i: (i, 0)),
            pl.BlockSpec((_RB, 64), lambda i: (i, 0)),
        ],
        out_shape=[
            jax.ShapeDtypeStruct((N, 64), jnp.float32),
            jax.ShapeDtypeStruct((N, 64), jnp.float32),
        ],
    )(xwlo, xwhi, degcnt)


def _gru1_body(acclo_ref, acchi_ref, ylo_ref, yhi_ref, deg_ref, prev_ref,
               wihlo_ref, wihhi_ref, whh_ref, bih_ref, bhh_ref,
               bclo_ref, bchi_ref, wc2_ref, emb_ref, y2_ref):
    dinv = _dinv_of(deg_ref[...])
    tlo = _leaky((acclo_ref[0] + acclo_ref[1] + ylo_ref[...])
                 * dinv[:, None] + bclo_ref[...])
    thi = _leaky((acchi_ref[0] + acchi_ref[1] + yhi_ref[...])
                 * dinv[:, None] + bchi_ref[...])
    dn = (((1,), (1,)), ((), ()))
    gi = (lax.dot_general(tlo, wihlo_ref[...], dn,
                          preferred_element_type=jnp.float32)
          + lax.dot_general(thi, wihhi_ref[...], dn,
                            preferred_element_type=jnp.float32)
          + bih_ref[...])
    gh = lax.dot_general(prev_ref[...], whh_ref[...], dn,
                         preferred_element_type=jnp.float32) + bhh_ref[...]
    r = jax.nn.sigmoid(gi[:, 0:128] + gh[:, 0:128])
    z = jax.nn.sigmoid(gi[:, 128:256] + gh[:, 128:256])
    n = jnp.tanh(gi[:, 256:384] + r * gh[:, 256:384])
    e1 = (1.0 - z) * n + z * prev_ref[...]
    emb_ref[...] = e1
    y2_ref[...] = jnp.dot(e1, wc2_ref[...],
                          preferred_element_type=jnp.float32) * dinv[:, None]


def _gru1_call(acclo, acchi, ylo, yhi, degcnt, prev1, wihlo, wihhi, whh1,
               bih1, bhh1, bclo, bchi, wc2):
    return pl.pallas_call(
        _gru1_body,
        grid=(_GRID,),
        in_specs=[
            pl.BlockSpec((NC, _RB, 64), lambda i: (0, i, 0)),
            pl.BlockSpec((NC, _RB, 64), lambda i: (0, i, 0)),
            pl.BlockSpec((_RB, 64), lambda i: (i, 0)),
            pl.BlockSpec((_RB, 64), lambda i: (i, 0)),
            pl.BlockSpec((NC, _RB, DW), lambda i: (0, i, 0)),
            pl.BlockSpec((_RB, 128), lambda i: (i, 0)),
            _full((384, 64)), _full((384, 64)), _full((384, 128)),
            _full((1, 384)), _full((1, 384)),
            _full((1, 64)), _full((1, 64)),
            _full((128, 64)),
        ],
        out_specs=[
            pl.BlockSpec((_RB, 128), lambda i: (i, 0)),
            pl.BlockSpec((_RB, 64), lambda i: (i, 0)),
        ],
        out_shape=[
            jax.ShapeDtypeStruct((N, 128), jnp.float32),
            jax.ShapeDtypeStruct((N, 64), jnp.float32),
        ],
    )(acclo, acchi, ylo, yhi, degcnt, prev1, wihlo, wihhi, whh1,
      bih1, bhh1, bclo, bchi, wc2)


def _gru2_body(acc_ref, y_ref, deg_ref, prev_ref,
               wir_ref, wiz_ref, win_ref, whr_ref, whz_ref, whn_ref,
               bir_ref, biz_ref, bin_ref, bhr_ref, bhz_ref, bhn_ref,
               bc_ref, wpq_ref, emb_ref, pq_ref):
    dinv = _dinv_of(deg_ref[...])
    agg = acc_ref[0] + acc_ref[1] + y_ref[...]
    t = _leaky(agg * dinv[:, None] + bc_ref[...])
    prev = prev_ref[...]
    dn = (((1,), (1,)), ((), ()))
    i_r = lax.dot_general(t, wir_ref[...], dn,
                          preferred_element_type=jnp.float32) + bir_ref[...]
    i_z = lax.dot_general(t, wiz_ref[...], dn,
                          preferred_element_type=jnp.float32) + biz_ref[...]
    i_n = lax.dot_general(t, win_ref[...], dn,
                          preferred_element_type=jnp.float32) + bin_ref[...]
    h_r = lax.dot_general(prev, whr_ref[...], dn,
                          preferred_element_type=jnp.float32) + bhr_ref[...]
    h_z = lax.dot_general(prev, whz_ref[...], dn,
                          preferred_element_type=jnp.float32) + bhz_ref[...]
    h_n = lax.dot_general(prev, whn_ref[...], dn,
                          preferred_element_type=jnp.float32) + bhn_ref[...]
    r = jax.nn.sigmoid(i_r + h_r)
    z = jax.nn.sigmoid(i_z + h_z)
    n = jnp.tanh(i_n + r * h_n)
    e2 = (1.0 - z) * n + z * prev
    emb_ref[...] = e2
    pq_ref[...] = jnp.dot(e2, wpq_ref[...], preferred_element_type=jnp.float32)


def _gru2_call(acc2, y2, degcnt, prev2, wir, wiz, win, whr, whz, whn,
               bir, biz, bin_, bhr, bhz, bhn, bc2, wpq):
    return pl.pallas_call(
        _gru2_body,
        grid=(_GRID,),
        in_specs=[
            pl.BlockSpec((NC, _RB, 64), lambda i: (0, i, 0)),
            pl.BlockSpec((_RB, 64), lambda i: (i, 0)),
            pl.BlockSpec((NC, _RB, DW), lambda i: (0, i, 0)),
            pl.BlockSpec((_RB, 64), lambda i: (i, 0)),
            _full((64, 64)), _full((64, 64)), _full((64, 64)),
            _full((64, 64)), _full((64, 64)), _full((64, 64)),
            _full((1, 64)), _full((1, 64)), _full((1, 64)),
            _full((1, 64)), _full((1, 64)), _full((1, 64)),
            _full((1, 64)), _full((64, 2)),
        ],
        out_specs=[
            pl.BlockSpec((_RB, 64), lambda i: (i, 0)),
            pl.BlockSpec((_RB, 2), lambda i: (i, 0)),
        ],
        out_shape=[
            jax.ShapeDtypeStruct((N, 64), jnp.float32),
            jax.ShapeDtypeStruct((N, 2), jnp.float32),
        ],
    )(acc2, y2, degcnt, prev2, wir, wiz, win, whr, whz, whn,
      bir, biz, bin_, bhr, bhz, bhn, bc2, wpq)


_ERows = E // 128
_ERB = _ERows // 10


def _r_body(attr_ref, wr_ref, bp_ref, o_ref):
    o_ref[...] = jnp.dot(attr_ref[...], wr_ref[...],
                         preferred_element_type=jnp.float32) + bp_ref[...]


def _r_call(attr128, wr_kron, bp):
    return pl.pallas_call(
        _r_body,
        grid=(1,),
        in_specs=[
            pl.BlockSpec((_ERows, 2048), lambda i: (0, 0)),
            _full((2048, 128)), _full((1, 1)),
        ],
        out_specs=pl.BlockSpec((_ERows, 128), lambda i: (0, 0)),
        out_shape=jax.ShapeDtypeStruct((_ERows, 128), jnp.float32),
    )(attr128, wr_kron, bp)


def kernel(x, edge_index, edge_label_index, edge_attr,
           W_pre1, b_pre1, W_pre2, b_pre2,
           W_conv1, b_conv1, W_conv2, b_conv2,
           wih1, whh1, bih1, bhh1,
           wih2, whh2, bih2, bhh2,
           W_post, b_post, prev_emb1, prev_emb2):
    src3 = edge_index[0].astype(jnp.int32).reshape(NW, NCH, C)
    dst3 = edge_index[1].astype(jnp.int32).reshape(NW, NCH, C)
    e0 = edge_label_index[0].astype(jnp.int32)
    e1 = edge_label_index[1].astype(jnp.int32)

    onesd = jnp.ones((C, DW), jnp.float32)
    zd = jnp.zeros((NPAD, DW), jnp.float32)
    z64 = jnp.zeros((NPAD, 64), jnp.float32)

    src3, dst3, e0, e1 = lax.optimization_barrier((src3, dst3, e0, e1))

    xwlo, xwhi = _pre_call(x, W_pre1, b_pre1.reshape(1, -1),
                           W_pre2, b_pre2.reshape(1, -1),
                           W_conv1[:, 0:64], W_conv1[:, 64:128])
    degcnt = _deg_call()(dst3, onesd, zd)
    wr_kron = jnp.kron(jnp.eye(128, dtype=jnp.float32), W_post[128:144])
    r = _r_call(edge_attr.reshape(_ERows, 2048), wr_kron,
                b_post.reshape(1, 1))
    ylo, yhi = _y1_call(xwlo, xwhi, degcnt)
    scat = _scatter_call()
    acclo = scat(ylo, src3, dst3, z64)
    acchi = scat(yhi, src3, dst3, z64)
    acclo, acchi, r = lax.optimization_barrier((acclo, acchi, r))
    emb1, y2 = _gru1_call(acclo, acchi, ylo, yhi, degcnt, prev_emb1,
                          wih1[:, 0:64], wih1[:, 64:128], whh1,
                          bih1.reshape(1, -1), bhh1.reshape(1, -1),
                          b_conv1[0:64].reshape(1, -1),
                          b_conv1[64:128].reshape(1, -1), W_conv2)
    acc2 = scat(y2, src3, dst3, z64)
    wpq = jnp.concatenate([W_post[0:64], W_post[64:128]], axis=1)
    emb2, pq = _gru2_call(acc2, y2, degcnt, prev_emb2,
                          wih2[0:64], wih2[64:128], wih2[128:192],
                          whh2[0:64], whh2[64:128], whh2[128:192],
                          bih2[0:64].reshape(1, -1),
                          bih2[64:128].reshape(1, -1),
                          bih2[128:192].reshape(1, -1),
                          bhh2[0:64].reshape(1, -1),
                          bhh2[64:128].reshape(1, -1),
                          bhh2[128:192].reshape(1, -1),
                          b_conv2.reshape(1, -1), wpq)
    out = _edge_call()(pq.reshape(2 * N), e0, e1, r.reshape(E))
    return out, emb1, emb2


# speedup vs baseline: 24.7273x; 1.0541x over previous
import functools

import jax
import jax.numpy as jnp
from jax import lax
from jax.experimental import pallas as pl
from jax.experimental.pallas import tpu as pltpu
from jax.experimental.pallas import tpu_sc as plsc

N = 10000
E = 320000
NC = 2
NS = 16
NW = NC * NS
EPW = E // NW
C = 80
NCH = EPW // C
NPAD = 10112
RPT = NPAD // NS
DW = 16


@functools.lru_cache(maxsize=None)
def _sc_mesh():
    return plsc.VectorSubcoreMesh(core_axis_name="c", subcore_axis_name="s",
                                  num_cores=NC, num_subcores=NS)


def _leaky(v):
    return jnp.where(v >= 0, v, 0.01 * v)


def _deg_body(ei_hbm, ones_hbm, zeros_hbm, degcnt_hbm, dst_v, ones_v, acc_sh):
    c = lax.axis_index("c")
    s = lax.axis_index("s")
    wid = c * NS + s
    pltpu.sync_copy(ei_hbm.at[1, wid], dst_v)
    pltpu.sync_copy(ones_hbm, ones_v)
    pltpu.sync_copy(zeros_hbm.at[pl.ds(s * RPT, RPT)],
                    acc_sh.at[pl.ds(s * RPT, RPT)])
    plsc.subcore_barrier()

    def body(j, carry):
        pltpu.sync_copy(ones_v, acc_sh.at[dst_v.at[j]], add=True)
        return carry

    lax.fori_loop(0, NCH, body, 0)
    plsc.subcore_barrier()
    pltpu.sync_copy(acc_sh.at[pl.ds(s * RPT, RPT)],
                    degcnt_hbm.at[c, pl.ds(s * RPT, RPT)])


@functools.lru_cache(maxsize=None)
def _deg_call():
    return pl.kernel(
        _deg_body,
        out_type=jax.ShapeDtypeStruct((NC, NPAD, DW), jnp.float32),
        mesh=_sc_mesh(),
        compiler_params=pltpu.CompilerParams(use_tc_tiling_on_sc=False),
        scratch_types=[
            pltpu.VMEM((NCH, C), jnp.int32),
            pltpu.VMEM((C, DW), jnp.float32),
            pltpu.VMEM_SHARED((NPAD, DW), jnp.float32),
        ],
    )


def _scatter_body(y_hbm, ei_hbm, zeros_hbm, acc_hbm,
                  src_v, dst_v, rows0, rows1, acc_sh, sem0, sem1):
    c = lax.axis_index("c")
    s = lax.axis_index("s")
    wid = c * NS + s
    pltpu.sync_copy(ei_hbm.at[0, wid], src_v)
    pltpu.sync_copy(ei_hbm.at[1, wid], dst_v)
    pltpu.sync_copy(zeros_hbm.at[pl.ds(s * RPT, RPT)],
                    acc_sh.at[pl.ds(s * RPT, RPT)])
    plsc.subcore_barrier()

    pltpu.async_copy(y_hbm.at[src_v.at[0]], rows0, sem0)

    def body(k, carry):
        j0 = 2 * k
        j1 = j0 + 1
        j2 = j0 + 2
        pltpu.async_copy(y_hbm.at[src_v.at[j1]], rows1, sem1)
        pltpu.make_async_copy(y_hbm.at[src_v.at[j0]], rows0, sem0).wait()
        pltpu.sync_copy(rows0, acc_sh.at[dst_v.at[j0]], add=True)
        pltpu.async_copy(y_hbm.at[src_v.at[j2]], rows0, sem0)
        pltpu.make_async_copy(y_hbm.at[src_v.at[j1]], rows1, sem1).wait()
        pltpu.sync_copy(rows1, acc_sh.at[dst_v.at[j1]], add=True)
        return carry

    lax.fori_loop(0, (NCH - 1) // 2, body, 0)
    pltpu.make_async_copy(y_hbm.at[src_v.at[NCH - 1]], rows0, sem0).wait()
    pltpu.sync_copy(rows0, acc_sh.at[dst_v.at[NCH - 1]], add=True)
    plsc.subcore_barrier()
    pltpu.sync_copy(acc_sh.at[pl.ds(s * RPT, RPT)],
                    acc_hbm.at[c, pl.ds(s * RPT, RPT)])


@functools.lru_cache(maxsize=None)
def _scatter_call():
    return pl.kernel(
        _scatter_body,
        out_type=jax.ShapeDtypeStruct((NC, NPAD, 64), jnp.float32),
        mesh=_sc_mesh(),
        compiler_params=pltpu.CompilerParams(use_tc_tiling_on_sc=False),
        scratch_types=[
            pltpu.VMEM((NCH, C), jnp.int32),
            pltpu.VMEM((NCH, C), jnp.int32),
            pltpu.VMEM((C, 64), jnp.float32),
            pltpu.VMEM((C, 64), jnp.float32),
            pltpu.VMEM_SHARED((NPAD, 64), jnp.float32),
            pltpu.SemaphoreType.DMA,
            pltpu.SemaphoreType.DMA,
        ],
    )


def _edge_body(pq_hbm, e0_hbm, e1_hbm, r_hbm, out_hbm,
               pq_v, e0_v, e1_v, r_v, o_v):
    c = lax.axis_index("c")
    s = lax.axis_index("s")
    wid = c * NS + s
    base = wid * EPW
    pltpu.sync_copy(pq_hbm, pq_v)
    pltpu.sync_copy(e0_hbm.at[pl.ds(base, EPW)], e0_v)
    pltpu.sync_copy(e1_hbm.at[pl.ds(base, EPW)], e1_v)
    pltpu.sync_copy(r_hbm.at[pl.ds(base, EPW)], r_v)

    def body(i, carry):
        sl = pl.ds(i * 16, 16)
        si = e0_v[sl]
        di = e1_v[sl]
        gp = plsc.load_gather(pq_v, [si * 2])
        gq = plsc.load_gather(pq_v, [di * 2 + 1])
        o_v[sl] = gp + gq + r_v[sl]
        return carry

    lax.fori_loop(0, EPW // 16, body, 0)
    pltpu.sync_copy(o_v, out_hbm.at[pl.ds(base, EPW)])


@functools.lru_cache(maxsize=None)
def _edge_call():
    return pl.kernel(
        _edge_body,
        out_type=jax.ShapeDtypeStruct((E,), jnp.float32),
        mesh=_sc_mesh(),
        compiler_params=pltpu.CompilerParams(needs_layout_passes=False),
        scratch_types=[
            pltpu.VMEM((2 * N,), jnp.float32),
            pltpu.VMEM((EPW,), jnp.int32),
            pltpu.VMEM((EPW,), jnp.int32),
            pltpu.VMEM((EPW,), jnp.float32),
            pltpu.VMEM((EPW,), jnp.float32),
        ],
    )


_RB = 1000
_GRID = N // _RB


def _full(shape):
    nd = len(shape)
    return pl.BlockSpec(shape, lambda i, _n=nd: (0,) * _n)


def _pre_body(x_ref, w1_ref, b1_ref, w2_ref, b2_ref, wclo_ref, wchi_ref,
              olo_ref, ohi_ref):
    h = jnp.dot(x_ref[...], w1_ref[...], preferred_element_type=jnp.float32)
    h = _leaky(h + b1_ref[...])
    h = jnp.dot(h, w2_ref[...], preferred_element_type=jnp.float32)
    h = _leaky(h + b2_ref[...])
    olo_ref[...] = jnp.dot(h, wclo_ref[...], preferred_element_type=jnp.float32)
    ohi_ref[...] = jnp.dot(h, wchi_ref[...], preferred_element_type=jnp.float32)


def _pre_call(x, w1, b1, w2, b2, wclo, wchi):
    return pl.pallas_call(
        _pre_body,
        grid=(_GRID,),
        in_specs=[
            pl.BlockSpec((_RB, 128), lambda i: (i, 0)),
            _full((128, 256)), _full((1, 256)),
            _full((256, 128)), _full((1, 128)),
            _full((128, 64)), _full((128, 64)),
        ],
        out_specs=[
            pl.BlockSpec((_RB, 64), lambda i: (i, 0)),
            pl.BlockSpec((_RB, 64), lambda i: (i, 0)),
        ],
        out_shape=[
            jax.ShapeDtypeStruct((N, 64), jnp.float32),
            jax.ShapeDtypeStruct((N, 64), jnp.float32),
        ],
    )(x, w1, b1, w2, b2, wclo, wchi)


def _dinv_of(deg_blk):
    return lax.rsqrt(1.0 + jnp.sum(deg_blk, axis=(0, 2)) * (1.0 / DW))


def _y1_body(xwlo_ref, xwhi_ref, deg_ref, olo_ref, ohi_ref):
    dinv = _dinv_of(deg_ref[...])
    olo_ref[...] = xwlo_ref[...] * dinv[:, None]
    ohi_ref[...] = xwhi_ref[...] * dinv[:, None]


def _y1_call(xwlo, xwhi, degcnt):
    return pl.pallas_call(
        _y1_body,
        grid=(_GRID,),
        in_specs=[
            pl.BlockSpec((_RB, 64), lambda i: (i, 0)),
            pl.BlockSpec((_RB, 64), lambda i: (i, 0)),
            pl.BlockSpec((NC, _RB, DW), lambda i: (0, i, 0)),
        ],
        out_specs=[
            pl.BlockSpec((_RB, 64), lambda i: (i, 0)),
            pl.BlockSpec((_RB, 64), lambda i: (i, 0)),
        ],
        out_shape=[
            jax.ShapeDtypeStruct((N, 64), jnp.float32),
            jax.ShapeDtypeStruct((N, 64), jnp.float32),
        ],
    )(xwlo, xwhi, degcnt)


def _gru1_body(acclo_ref, acchi_ref, ylo_ref, yhi_ref, deg_ref, prev_ref,
               wihlo_ref, wihhi_ref, whh_ref, bih_ref, bhh_ref,
               bclo_ref, bchi_ref, wc2_ref, emb_ref, y2_ref):
    dinv = _dinv_of(deg_ref[...])
    tlo = _leaky((acclo_ref[0] + acclo_ref[1] + ylo_ref[...])
                 * dinv[:, None] + bclo_ref[...])
    thi = _leaky((acchi_ref[0] + acchi_ref[1] + yhi_ref[...])
                 * dinv[:, None] + bchi_ref[...])
    dn = (((1,), (1,)), ((), ()))
    gi = (lax.dot_general(tlo, wihlo_ref[...], dn,
                          preferred_element_type=jnp.float32)
          + lax.dot_general(thi, wihhi_ref[...], dn,
                            preferred_element_type=jnp.float32)
          + bih_ref[...])
    gh = lax.dot_general(prev_ref[...], whh_ref[...], dn,
                         preferred_element_type=jnp.float32) + bhh_ref[...]
    r = jax.nn.sigmoid(gi[:, 0:128] + gh[:, 0:128])
    z = jax.nn.sigmoid(gi[:, 128:256] + gh[:, 128:256])
    n = jnp.tanh(gi[:, 256:384] + r * gh[:, 256:384])
    e1 = (1.0 - z) * n + z * prev_ref[...]
    emb_ref[...] = e1
    y2_ref[...] = jnp.dot(e1, wc2_ref[...],
                          preferred_element_type=jnp.float32) * dinv[:, None]


def _gru1_call(acclo, acchi, ylo, yhi, degcnt, prev1, wihlo, wihhi, whh1,
               bih1, bhh1, bclo, bchi, wc2):
    return pl.pallas_call(
        _gru1_body,
        grid=(_GRID,),
        in_specs=[
            pl.BlockSpec((NC, _RB, 64), lambda i: (0, i, 0)),
            pl.BlockSpec((NC, _RB, 64), lambda i: (0, i, 0)),
            pl.BlockSpec((_RB, 64), lambda i: (i, 0)),
            pl.BlockSpec((_RB, 64), lambda i: (i, 0)),
            pl.BlockSpec((NC, _RB, DW), lambda i: (0, i, 0)),
            pl.BlockSpec((_RB, 128), lambda i: (i, 0)),
            _full((384, 64)), _full((384, 64)), _full((384, 128)),
            _full((1, 384)), _full((1, 384)),
            _full((1, 64)), _full((1, 64)),
            _full((128, 64)),
        ],
        out_specs=[
            pl.BlockSpec((_RB, 128), lambda i: (i, 0)),
            pl.BlockSpec((_RB, 64), lambda i: (i, 0)),
        ],
        out_shape=[
            jax.ShapeDtypeStruct((N, 128), jnp.float32),
            jax.ShapeDtypeStruct((N, 64), jnp.float32),
        ],
    )(acclo, acchi, ylo, yhi, degcnt, prev1, wihlo, wihhi, whh1,
      bih1, bhh1, bclo, bchi, wc2)


def _gru2_body(acc_ref, y_ref, deg_ref, prev_ref,
               wir_ref, wiz_ref, win_ref, whr_ref, whz_ref, whn_ref,
               bir_ref, biz_ref, bin_ref, bhr_ref, bhz_ref, bhn_ref,
               bc_ref, wpq_ref, emb_ref, pq_ref):
    dinv = _dinv_of(deg_ref[...])
    agg = acc_ref[0] + acc_ref[1] + y_ref[...]
    t = _leaky(agg * dinv[:, None] + bc_ref[...])
    prev = prev_ref[...]
    dn = (((1,), (1,)), ((), ()))
    i_r = lax.dot_general(t, wir_ref[...], dn,
                          preferred_element_type=jnp.float32) + bir_ref[...]
    i_z = lax.dot_general(t, wiz_ref[...], dn,
                          preferred_element_type=jnp.float32) + biz_ref[...]
    i_n = lax.dot_general(t, win_ref[...], dn,
                          preferred_element_type=jnp.float32) + bin_ref[...]
    h_r = lax.dot_general(prev, whr_ref[...], dn,
                          preferred_element_type=jnp.float32) + bhr_ref[...]
    h_z = lax.dot_general(prev, whz_ref[...], dn,
                          preferred_element_type=jnp.float32) + bhz_ref[...]
    h_n = lax.dot_general(prev, whn_ref[...], dn,
                          preferred_element_type=jnp.float32) + bhn_ref[...]
    r = jax.nn.sigmoid(i_r + h_r)
    z = jax.nn.sigmoid(i_z + h_z)
    n = jnp.tanh(i_n + r * h_n)
    e2 = (1.0 - z) * n + z * prev
    emb_ref[...] = e2
    pq_ref[...] = jnp.dot(e2, wpq_ref[...], preferred_element_type=jnp.float32)


def _gru2_call(acc2, y2, degcnt, prev2, wir, wiz, win, whr, whz, whn,
               bir, biz, bin_, bhr, bhz, bhn, bc2, wpq):
    return pl.pallas_call(
        _gru2_body,
        grid=(_GRID,),
        in_specs=[
            pl.BlockSpec((NC, _RB, 64), lambda i: (0, i, 0)),
            pl.BlockSpec((_RB, 64), lambda i: (i, 0)),
            pl.BlockSpec((NC, _RB, DW), lambda i: (0, i, 0)),
            pl.BlockSpec((_RB, 64), lambda i: (i, 0)),
            _full((64, 64)), _full((64, 64)), _full((64, 64)),
            _full((64, 64)), _full((64, 64)), _full((64, 64)),
            _full((1, 64)), _full((1, 64)), _full((1, 64)),
            _full((1, 64)), _full((1, 64)), _full((1, 64)),
            _full((1, 64)), _full((64, 2)),
        ],
        out_specs=[
            pl.BlockSpec((_RB, 64), lambda i: (i, 0)),
            pl.BlockSpec((_RB, 2), lambda i: (i, 0)),
        ],
        out_shape=[
            jax.ShapeDtypeStruct((N, 64), jnp.float32),
            jax.ShapeDtypeStruct((N, 2), jnp.float32),
        ],
    )(acc2, y2, degcnt, prev2, wir, wiz, win, whr, whz, whn,
      bir, biz, bin_, bhr, bhz, bhn, bc2, wpq)


_ERows = E // 128
_ERB = _ERows // 10


def _r_body(attr_ref, wr_ref, bp_ref, o_ref):
    o_ref[...] = jnp.dot(attr_ref[...], wr_ref[...],
                         preferred_element_type=jnp.float32) + bp_ref[...]


def _r_call(attr128, wr_kron, bp):
    return pl.pallas_call(
        _r_body,
        grid=(1,),
        in_specs=[
            pl.BlockSpec((_ERows, 2048), lambda i: (0, 0)),
            _full((2048, 128)), _full((1, 1)),
        ],
        out_specs=pl.BlockSpec((_ERows, 128), lambda i: (0, 0)),
        out_shape=jax.ShapeDtypeStruct((_ERows, 128), jnp.float32),
    )(attr128, wr_kron, bp)


def kernel(x, edge_index, edge_label_index, edge_attr,
           W_pre1, b_pre1, W_pre2, b_pre2,
           W_conv1, b_conv1, W_conv2, b_conv2,
           wih1, whh1, bih1, bhh1,
           wih2, whh2, bih2, bhh2,
           W_post, b_post, prev_emb1, prev_emb2):
    ei3 = edge_index.astype(jnp.int32).reshape(2, NW, NCH, C)
    e0 = edge_label_index[0].astype(jnp.int32)
    e1 = edge_label_index[1].astype(jnp.int32)

    onesd = jnp.ones((C, DW), jnp.float32)
    zd = jnp.zeros((NPAD, DW), jnp.float32)
    z64 = jnp.zeros((NPAD, 64), jnp.float32)

    ei3 = lax.optimization_barrier(ei3)

    xwlo, xwhi = _pre_call(x, W_pre1, b_pre1.reshape(1, -1),
                           W_pre2, b_pre2.reshape(1, -1),
                           W_conv1[:, 0:64], W_conv1[:, 64:128])
    degcnt = _deg_call()(ei3, onesd, zd)
    wr_kron = jnp.kron(jnp.eye(128, dtype=jnp.float32), W_post[128:144])
    r = _r_call(edge_attr.reshape(_ERows, 2048), wr_kron,
                b_post.reshape(1, 1))
    ylo, yhi = _y1_call(xwlo, xwhi, degcnt)
    scat = _scatter_call()
    acclo = scat(ylo, ei3, z64)
    acchi = scat(yhi, ei3, z64)
    acclo, acchi, r, e0, e1 = lax.optimization_barrier(
        (acclo, acchi, r, e0, e1))
    emb1, y2 = _gru1_call(acclo, acchi, ylo, yhi, degcnt, prev_emb1,
                          wih1[:, 0:64], wih1[:, 64:128], whh1,
                          bih1.reshape(1, -1), bhh1.reshape(1, -1),
                          b_conv1[0:64].reshape(1, -1),
                          b_conv1[64:128].reshape(1, -1), W_conv2)
    acc2 = scat(y2, ei3, z64)
    wpq = jnp.concatenate([W_post[0:64], W_post[64:128]], axis=1)
    emb2, pq = _gru2_call(acc2, y2, degcnt, prev_emb2,
                          wih2[0:64], wih2[64:128], wih2[128:192],
                          whh2[0:64], whh2[64:128], whh2[128:192],
                          bih2[0:64].reshape(1, -1),
                          bih2[64:128].reshape(1, -1),
                          bih2[128:192].reshape(1, -1),
                          bhh2[0:64].reshape(1, -1),
                          bhh2[64:128].reshape(1, -1),
                          bhh2[128:192].reshape(1, -1),
                          b_conv2.reshape(1, -1), wpq)
    out = _edge_call()(pq.reshape(2 * N), e0, e1, r.reshape(E))
    return out, emb1, emb2


# speedup vs baseline: 26.4614x; 1.0701x over previous
import functools

import jax
import jax.numpy as jnp
from jax import lax
from jax.experimental import pallas as pl
from jax.experimental.pallas import tpu as pltpu
from jax.experimental.pallas import tpu_sc as plsc

N = 10000
E = 320000
NC = 2
NS = 16
NW = NC * NS
EPW = E // NW
C = 128
EP = 327680
EPW2 = EP // NW
NCH = EPW2 // C
NPAD = 10112
RPT = NPAD // NS
DW = 8


@functools.lru_cache(maxsize=None)
def _sc_mesh():
    return plsc.VectorSubcoreMesh(core_axis_name="c", subcore_axis_name="s",
                                  num_cores=NC, num_subcores=NS)


def _leaky(v):
    return jnp.where(v >= 0, v, 0.01 * v)


def _deg_body(ei_hbm, ones_hbm, zeros_hbm, degcnt_hbm, dst_v, ones_v, acc_sh):
    c = lax.axis_index("c")
    s = lax.axis_index("s")
    wid = c * NS + s
    pltpu.sync_copy(ei_hbm.at[1, wid], dst_v)
    pltpu.sync_copy(ones_hbm, ones_v)
    pltpu.sync_copy(zeros_hbm.at[pl.ds(s * RPT, RPT)],
                    acc_sh.at[pl.ds(s * RPT, RPT)])
    plsc.subcore_barrier()

    def body(j, carry):
        pltpu.sync_copy(ones_v, acc_sh.at[dst_v.at[j]], add=True)
        return carry

    lax.fori_loop(0, NCH, body, 0)
    plsc.subcore_barrier()
    pltpu.sync_copy(acc_sh.at[pl.ds(s * RPT, RPT)],
                    degcnt_hbm.at[c, pl.ds(s * RPT, RPT)])


@functools.lru_cache(maxsize=None)
def _deg_call():
    return pl.kernel(
        _deg_body,
        out_type=jax.ShapeDtypeStruct((NC, NPAD, DW), jnp.float32),
        mesh=_sc_mesh(),
        compiler_params=pltpu.CompilerParams(use_tc_tiling_on_sc=False),
        scratch_types=[
            pltpu.VMEM((NCH, C), jnp.int32),
            pltpu.VMEM((C, DW), jnp.float32),
            pltpu.VMEM_SHARED((NPAD, DW), jnp.float32),
        ],
    )


def _scatter_body(y_hbm, ei_hbm, zeros_hbm, acc_hbm,
                  src_v, dst_v, rows0, rows1, acc_sh, sem0, sem1):
    c = lax.axis_index("c")
    s = lax.axis_index("s")
    wid = c * NS + s
    pltpu.sync_copy(ei_hbm.at[0, wid], src_v)
    pltpu.sync_copy(ei_hbm.at[1, wid], dst_v)
    pltpu.sync_copy(zeros_hbm.at[pl.ds(s * RPT, RPT)],
                    acc_sh.at[pl.ds(s * RPT, RPT)])
    plsc.subcore_barrier()

    pltpu.async_copy(y_hbm.at[src_v.at[0]], rows0, sem0)

    def body(k, carry):
        j0 = 2 * k
        j1 = j0 + 1
        j2 = j0 + 2
        pltpu.async_copy(y_hbm.at[src_v.at[j1]], rows1, sem1)
        pltpu.make_async_copy(y_hbm.at[src_v.at[j0]], rows0, sem0).wait()
        pltpu.sync_copy(rows0, acc_sh.at[dst_v.at[j0]], add=True)
        pltpu.async_copy(y_hbm.at[src_v.at[j2]], rows0, sem0)
        pltpu.make_async_copy(y_hbm.at[src_v.at[j1]], rows1, sem1).wait()
        pltpu.sync_copy(rows1, acc_sh.at[dst_v.at[j1]], add=True)
        return carry

    lax.fori_loop(0, (NCH - 2) // 2, body, 0)
    pltpu.async_copy(y_hbm.at[src_v.at[NCH - 1]], rows1, sem1)
    pltpu.make_async_copy(y_hbm.at[src_v.at[NCH - 2]], rows0, sem0).wait()
    pltpu.sync_copy(rows0, acc_sh.at[dst_v.at[NCH - 2]], add=True)
    pltpu.make_async_copy(y_hbm.at[src_v.at[NCH - 1]], rows1, sem1).wait()
    pltpu.sync_copy(rows1, acc_sh.at[dst_v.at[NCH - 1]], add=True)
    plsc.subcore_barrier()
    pltpu.sync_copy(acc_sh.at[pl.ds(s * RPT, RPT)],
                    acc_hbm.at[c, pl.ds(s * RPT, RPT)])


@functools.lru_cache(maxsize=None)
def _scatter_call():
    return pl.kernel(
        _scatter_body,
        out_type=jax.ShapeDtypeStruct((NC, NPAD, 64), jnp.float32),
        mesh=_sc_mesh(),
        compiler_params=pltpu.CompilerParams(use_tc_tiling_on_sc=False),
        scratch_types=[
            pltpu.VMEM((NCH, C), jnp.int32),
            pltpu.VMEM((NCH, C), jnp.int32),
            pltpu.VMEM((C, 64), jnp.float32),
            pltpu.VMEM((C, 64), jnp.float32),
            pltpu.VMEM_SHARED((NPAD, 64), jnp.float32),
            pltpu.SemaphoreType.DMA,
            pltpu.SemaphoreType.DMA,
        ],
    )


def _edge_body(pq_hbm, e0_hbm, e1_hbm, r_hbm, out_hbm,
               pq_v, e0_v, e1_v, r_v, o_v):
    c = lax.axis_index("c")
    s = lax.axis_index("s")
    wid = c * NS + s
    base = wid * EPW
    pltpu.sync_copy(pq_hbm, pq_v)
    pltpu.sync_copy(e0_hbm.at[pl.ds(base, EPW)], e0_v)
    pltpu.sync_copy(e1_hbm.at[pl.ds(base, EPW)], e1_v)
    pltpu.sync_copy(r_hbm.at[pl.ds(base, EPW)], r_v)

    def body(i, carry):
        sl = pl.ds(i * 16, 16)
        si = e0_v[sl]
        di = e1_v[sl]
        gp = plsc.load_gather(pq_v, [si * 2])
        gq = plsc.load_gather(pq_v, [di * 2 + 1])
        o_v[sl] = gp + gq + r_v[sl]
        return carry

    lax.fori_loop(0, EPW // 16, body, 0)
    pltpu.sync_copy(o_v, out_hbm.at[pl.ds(base, EPW)])


@functools.lru_cache(maxsize=None)
def _edge_call():
    return pl.kernel(
        _edge_body,
        out_type=jax.ShapeDtypeStruct((E,), jnp.float32),
        mesh=_sc_mesh(),
        compiler_params=pltpu.CompilerParams(needs_layout_passes=False),
        scratch_types=[
            pltpu.VMEM((2 * N,), jnp.float32),
            pltpu.VMEM((EPW,), jnp.int32),
            pltpu.VMEM((EPW,), jnp.int32),
            pltpu.VMEM((EPW,), jnp.float32),
            pltpu.VMEM((EPW,), jnp.float32),
        ],
    )


_RB = 1000
_GRID = N // _RB


def _full(shape):
    nd = len(shape)
    return pl.BlockSpec(shape, lambda i, _n=nd: (0,) * _n)


def _pre_body(x_ref, w1_ref, b1_ref, w2_ref, b2_ref, wclo_ref, wchi_ref,
              olo_ref, ohi_ref):
    h = jnp.dot(x_ref[...], w1_ref[...], preferred_element_type=jnp.float32)
    h = _leaky(h + b1_ref[...])
    h = jnp.dot(h, w2_ref[...], preferred_element_type=jnp.float32)
    h = _leaky(h + b2_ref[...])
    olo_ref[...] = jnp.dot(h, wclo_ref[...], preferred_element_type=jnp.float32)
    ohi_ref[...] = jnp.dot(h, wchi_ref[...], preferred_element_type=jnp.float32)


def _pre_call(x, w1, b1, w2, b2, wclo, wchi):
    return pl.pallas_call(
        _pre_body,
        grid=(_GRID,),
        in_specs=[
            pl.BlockSpec((_RB, 128), lambda i: (i, 0)),
            _full((128, 256)), _full((1, 256)),
            _full((256, 128)), _full((1, 128)),
            _full((128, 64)), _full((128, 64)),
        ],
        out_specs=[
            pl.BlockSpec((_RB, 64), lambda i: (i, 0)),
            pl.BlockSpec((_RB, 64), lambda i: (i, 0)),
        ],
        out_shape=[
            jax.ShapeDtypeStruct((N, 64), jnp.float32),
            jax.ShapeDtypeStruct((N, 64), jnp.float32),
        ],
    )(x, w1, b1, w2, b2, wclo, wchi)


def _dinv_of(deg_blk):
    return lax.rsqrt(1.0 + jnp.sum(deg_blk, axis=(0, 2)) * (1.0 / DW))


def _y1_body(xwlo_ref, xwhi_ref, deg_ref, olo_ref, ohi_ref):
    dinv = _dinv_of(deg_ref[...])
    olo_ref[...] = xwlo_ref[...] * dinv[:, None]
    ohi_ref[...] = xwhi_ref[...] * dinv[:, None]


def _y1_call(xwlo, xwhi, degcnt):
    return pl.pallas_call(
        _y1_body,
        grid=(_GRID,),
        in_specs=[
            pl.BlockSpec((_RB, 64), lambda i: (i, 0)),
            pl.BlockSpec((_RB, 64), lambda i: (i, 0)),
            pl.BlockSpec((NC, _RB, DW), lambda i: (0, i, 0)),
        ],
        out_specs=[
            pl.BlockSpec((_RB, 64), lambda i: (i, 0)),
            pl.BlockSpec((_RB, 64), lambda i: (i, 0)),
        ],
        out_shape=[
            jax.ShapeDtypeStruct((N, 64), jnp.float32),
            jax.ShapeDtypeStruct((N, 64), jnp.float32),
        ],
    )(xwlo, xwhi, degcnt)


def _gru1_body(acclo_ref, acchi_ref, ylo_ref, yhi_ref, deg_ref, prev_ref,
               wihlo_ref, wihhi_ref, whh_ref, bih_ref, bhh_ref,
               bclo_ref, bchi_ref, wc2_ref, emb_ref, y2_ref):
    dinv = _dinv_of(deg_ref[...])
    tlo = _leaky((acclo_ref[0] + acclo_ref[1] + ylo_ref[...])
                 * dinv[:, None] + bclo_ref[...])
    thi = _leaky((acchi_ref[0] + acchi_ref[1] + yhi_ref[...])
                 * dinv[:, None] + bchi_ref[...])
    dn = (((1,), (1,)), ((), ()))
    gi = (lax.dot_general(tlo, wihlo_ref[...], dn,
                          preferred_element_type=jnp.float32)
          + lax.dot_general(thi, wihhi_ref[...], dn,
                            preferred_element_type=jnp.float32)
          + bih_ref[...])
    gh = lax.dot_general(prev_ref[...], whh_ref[...], dn,
                         preferred_element_type=jnp.float32) + bhh_ref[...]
    r = jax.nn.sigmoid(gi[:, 0:128] + gh[:, 0:128])
    z = jax.nn.sigmoid(gi[:, 128:256] + gh[:, 128:256])
    n = jnp.tanh(gi[:, 256:384] + r * gh[:, 256:384])
    e1 = (1.0 - z) * n + z * prev_ref[...]
    emb_ref[...] = e1
    y2_ref[...] = jnp.dot(e1, wc2_ref[...],
                          preferred_element_type=jnp.float32) * dinv[:, None]


def _gru1_call(acclo, acchi, ylo, yhi, degcnt, prev1, wihlo, wihhi, whh1,
               bih1, bhh1, bclo, bchi, wc2):
    return pl.pallas_call(
        _gru1_body,
        grid=(_GRID,),
        in_specs=[
            pl.BlockSpec((NC, _RB, 64), lambda i: (0, i, 0)),
            pl.BlockSpec((NC, _RB, 64), lambda i: (0, i, 0)),
            pl.BlockSpec((_RB, 64), lambda i: (i, 0)),
            pl.BlockSpec((_RB, 64), lambda i: (i, 0)),
            pl.BlockSpec((NC, _RB, DW), lambda i: (0, i, 0)),
            pl.BlockSpec((_RB, 128), lambda i: (i, 0)),
            _full((384, 64)), _full((384, 64)), _full((384, 128)),
            _full((1, 384)), _full((1, 384)),
            _full((1, 64)), _full((1, 64)),
            _full((128, 64)),
        ],
        out_specs=[
            pl.BlockSpec((_RB, 128), lambda i: (i, 0)),
            pl.BlockSpec((_RB, 64), lambda i: (i, 0)),
        ],
        out_shape=[
            jax.ShapeDtypeStruct((N, 128), jnp.float32),
            jax.ShapeDtypeStruct((N, 64), jnp.float32),
        ],
    )(acclo, acchi, ylo, yhi, degcnt, prev1, wihlo, wihhi, whh1,
      bih1, bhh1, bclo, bchi, wc2)


def _gru2_body(acc_ref, y_ref, deg_ref, prev_ref,
               wir_ref, wiz_ref, win_ref, whr_ref, whz_ref, whn_ref,
               bir_ref, biz_ref, bin_ref, bhr_ref, bhz_ref, bhn_ref,
               bc_ref, wpq_ref, emb_ref, pq_ref):
    dinv = _dinv_of(deg_ref[...])
    agg = acc_ref[0] + acc_ref[1] + y_ref[...]
    t = _leaky(agg * dinv[:, None] + bc_ref[...])
    prev = prev_ref[...]
    dn = (((1,), (1,)), ((), ()))
    i_r = lax.dot_general(t, wir_ref[...], dn,
                          preferred_element_type=jnp.float32) + bir_ref[...]
    i_z = lax.dot_general(t, wiz_ref[...], dn,
                          preferred_element_type=jnp.float32) + biz_ref[...]
    i_n = lax.dot_general(t, win_ref[...], dn,
                          preferred_element_type=jnp.float32) + bin_ref[...]
    h_r = lax.dot_general(prev, whr_ref[...], dn,
                          preferred_element_type=jnp.float32) + bhr_ref[...]
    h_z = lax.dot_general(prev, whz_ref[...], dn,
                          preferred_element_type=jnp.float32) + bhz_ref[...]
    h_n = lax.dot_general(prev, whn_ref[...], dn,
                          preferred_element_type=jnp.float32) + bhn_ref[...]
    r = jax.nn.sigmoid(i_r + h_r)
    z = jax.nn.sigmoid(i_z + h_z)
    n = jnp.tanh(i_n + r * h_n)
    e2 = (1.0 - z) * n + z * prev
    emb_ref[...] = e2
    pq_ref[...] = jnp.dot(e2, wpq_ref[...], preferred_element_type=jnp.float32)


def _gru2_call(acc2, y2, degcnt, prev2, wir, wiz, win, whr, whz, whn,
               bir, biz, bin_, bhr, bhz, bhn, bc2, wpq):
    return pl.pallas_call(
        _gru2_body,
        grid=(_GRID,),
        in_specs=[
            pl.BlockSpec((NC, _RB, 64), lambda i: (0, i, 0)),
            pl.BlockSpec((_RB, 64), lambda i: (i, 0)),
            pl.BlockSpec((NC, _RB, DW), lambda i: (0, i, 0)),
            pl.BlockSpec((_RB, 64), lambda i: (i, 0)),
            _full((64, 64)), _full((64, 64)), _full((64, 64)),
            _full((64, 64)), _full((64, 64)), _full((64, 64)),
            _full((1, 64)), _full((1, 64)), _full((1, 64)),
            _full((1, 64)), _full((1, 64)), _full((1, 64)),
            _full((1, 64)), _full((64, 2)),
        ],
        out_specs=[
            pl.BlockSpec((_RB, 64), lambda i: (i, 0)),
            pl.BlockSpec((_RB, 2), lambda i: (i, 0)),
        ],
        out_shape=[
            jax.ShapeDtypeStruct((N, 64), jnp.float32),
            jax.ShapeDtypeStruct((N, 2), jnp.float32),
        ],
    )(acc2, y2, degcnt, prev2, wir, wiz, win, whr, whz, whn,
      bir, biz, bin_, bhr, bhz, bhn, bc2, wpq)


_ERows = E // 128
_ERB = _ERows // 10


def _r_body(attr_ref, wr_ref, bp_ref, o_ref):
    o_ref[...] = jnp.dot(attr_ref[...], wr_ref[...],
                         preferred_element_type=jnp.float32) + bp_ref[...]


def _r_call(attr128, wr_kron, bp):
    return pl.pallas_call(
        _r_body,
        grid=(1,),
        in_specs=[
            pl.BlockSpec((_ERows, 2048), lambda i: (0, 0)),
            _full((2048, 128)), _full((1, 1)),
        ],
        out_specs=pl.BlockSpec((_ERows, 128), lambda i: (0, 0)),
        out_shape=jax.ShapeDtypeStruct((_ERows, 128), jnp.float32),
    )(attr128, wr_kron, bp)


def kernel(x, edge_index, edge_label_index, edge_attr,
           W_pre1, b_pre1, W_pre2, b_pre2,
           W_conv1, b_conv1, W_conv2, b_conv2,
           wih1, whh1, bih1, bhh1,
           wih2, whh2, bih2, bhh2,
           W_post, b_post, prev_emb1, prev_emb2):
    npadrows = jnp.arange(EP - E, dtype=jnp.int32)
    pad = jnp.stack([npadrows % N, N + npadrows % (NPAD - N)])
    ei3 = jnp.concatenate([edge_index.astype(jnp.int32), pad],
                          axis=1).reshape(2, NW, NCH, C)
    e0 = edge_label_index[0].astype(jnp.int32)
    e1 = edge_label_index[1].astype(jnp.int32)

    onesd = jnp.ones((C, DW), jnp.float32)
    zd = jnp.zeros((NPAD, DW), jnp.float32)
    z64 = jnp.zeros((NPAD, 64), jnp.float32)

    ei3 = lax.optimization_barrier(ei3)

    xwlo, xwhi = _pre_call(x, W_pre1, b_pre1.reshape(1, -1),
                           W_pre2, b_pre2.reshape(1, -1),
                           W_conv1[:, 0:64], W_conv1[:, 64:128])
    degcnt = _deg_call()(ei3, onesd, zd)
    wr_kron = jnp.kron(jnp.eye(128, dtype=jnp.float32), W_post[128:144])
    r = _r_call(edge_attr.reshape(_ERows, 2048), wr_kron,
                b_post.reshape(1, 1))
    ylo, yhi = _y1_call(xwlo, xwhi, degcnt)
    scat = _scatter_call()
    acclo = scat(ylo, ei3, z64)
    acchi = scat(yhi, ei3, z64)
    acclo, acchi, r, e0, e1 = lax.optimization_barrier(
        (acclo, acchi, r, e0, e1))
    emb1, y2 = _gru1_call(acclo, acchi, ylo, yhi, degcnt, prev_emb1,
                          wih1[:, 0:64], wih1[:, 64:128], whh1,
                          bih1.reshape(1, -1), bhh1.reshape(1, -1),
                          b_conv1[0:64].reshape(1, -1),
                          b_conv1[64:128].reshape(1, -1), W_conv2)
    acc2 = scat(y2, ei3, z64)
    wpq = jnp.concatenate([W_post[0:64], W_post[64:128]], axis=1)
    emb2, pq = _gru2_call(acc2, y2, degcnt, prev_emb2,
                          wih2[0:64], wih2[64:128], wih2[128:192],
                          whh2[0:64], whh2[64:128], whh2[128:192],
                          bih2[0:64].reshape(1, -1),
                          bih2[64:128].reshape(1, -1),
                          bih2[128:192].reshape(1, -1),
                          bhh2[0:64].reshape(1, -1),
                          bhh2[64:128].reshape(1, -1),
                          bhh2[128:192].reshape(1, -1),
                          b_conv2.reshape(1, -1), wpq)
    out = _edge_call()(pq.reshape(2 * N), e0, e1, r.reshape(E))
    return out, emb1, emb2
